# baseline (speedup 1.0000x reference)
import functools

import jax
import jax.numpy as jnp
import numpy as np
from jax import lax
from jax.experimental import pallas as pl
from jax.experimental.pallas import tpu as pltpu
from jax.experimental.pallas import tpu_sc as plsc

N = 10000
E = 320000
EMBED = 16
HID = 128
HEADS = 8
DH = 16
VOCAB = 20000
L = 2
NEG = -1e30
BIG = 1e30

_MESH = plsc.VectorSubcoreMesh(core_axis_name="c", subcore_axis_name="s")
_SC_PARAMS = pltpu.CompilerParams(use_tc_tiling_on_sc=False)
_SC_PARAMS_NL = pltpu.CompilerParams(use_tc_tiling_on_sc=False,
                                     needs_layout_passes=False)
NW = 32
ROWS_PER_TILE = N // 16


def _wid():
    return lax.axis_index("c") * 16 + lax.axis_index("s")


def _sc_embed_gather(table, idx2d, n_chunks):
    @functools.partial(
        pl.kernel,
        out_type=jax.ShapeDtypeStruct((n_chunks * 1024, 16), jnp.float32),
        mesh=_MESH,
        compiler_params=_SC_PARAMS,
        scratch_types=[
            pltpu.VMEM((8, 128), jnp.int32),
            pltpu.VMEM((1024, 16), jnp.float32),
            pltpu.SemaphoreType.DMA,
        ],
    )
    def k(table_h, idx_h, out_h, idx_v, rows_v, sem):
        w = _wid()
        n_iter = (n_chunks + NW - 1) // NW

        def body(i, carry):
            t = w + i * NW

            @pl.when(t < n_chunks)
            def _():
                pltpu.sync_copy(idx_h.at[pl.ds(t * 8, 8)], idx_v)
                descs = [
                    pltpu.async_copy(
                        table_h.at[idx_v.at[j]],
                        rows_v.at[pl.ds(j * 128, 128)],
                        sem,
                    )
                    for j in range(8)
                ]
                for d in descs:
                    d.wait()
                pltpu.sync_copy(rows_v, out_h.at[pl.ds(t * 1024, 1024)])

            return carry

        lax.fori_loop(0, n_iter, body, 0)

    return k(table, idx2d)


def _sc_e1(q, k, e_l, dst2d, src2d):
    nch = E // 128

    @functools.partial(
        pl.kernel,
        out_type=(
            jax.ShapeDtypeStruct((E, 16), jnp.float32),
            jax.ShapeDtypeStruct((NW, 16), jnp.float32),
        ),
        mesh=_MESH,
        compiler_params=_SC_PARAMS_NL,
        scratch_types=[
            pltpu.VMEM((1, 128), jnp.int32),
            pltpu.VMEM((1, 128), jnp.int32),
            pltpu.VMEM((128, HID), jnp.float32),
            pltpu.VMEM((128, HID), jnp.float32),
            pltpu.VMEM((128, HID), jnp.float32),
            pltpu.VMEM((128, 16), jnp.float32),
            pltpu.VMEM((8, 16), jnp.float32),
            pltpu.VMEM((16,), jnp.float32),
            pltpu.SemaphoreType.DMA,
        ],
    )
    def k_(q_h, k_h, e_h, dst_h, src_h, lg_h, gmp_h,
           dbuf, sbuf, qb, kb, eb, lg, mxh, mxb, sem):
        w = _wid()
        n_iter = (nch + NW - 1) // NW
        iota = lax.iota(jnp.int32, 16)
        for h in range(HEADS):
            mxh[h] = jnp.full((16,), NEG, jnp.float32)

        def body(i, carry):
            t = w + i * NW

            @pl.when(t < nch)
            def _():
                pltpu.sync_copy(dst_h.at[pl.ds(t, 1)], dbuf)
                pltpu.sync_copy(src_h.at[pl.ds(t, 1)], sbuf)
                descs = [
                    pltpu.async_copy(q_h.at[dbuf.at[0]], qb, sem),
                    pltpu.async_copy(k_h.at[sbuf.at[0]], kb, sem),
                    pltpu.async_copy(e_h.at[pl.ds(t * 128, 128)], eb, sem),
                ]
                for d in descs:
                    d.wait()

                def grp(g, c2):
                    ri = g * 16 + iota
                    for h in range(HEADS):
                        acc = jnp.zeros((16,), jnp.float32)
                        for j in range(DH):
                            cs = jnp.full((16,), h * DH + j, jnp.int32)
                            qv = plsc.load_gather(qb, [ri, cs])
                            kv = plsc.load_gather(kb, [ri, cs])
                            ev = plsc.load_gather(eb, [ri, cs])
                            acc = acc + qv * (kv + ev)
                        acc = acc * 0.25
                        plsc.store_scatter(
                            lg, [ri, jnp.full((16,), h, jnp.int32)], acc)
                        mxh[h] = jnp.maximum(mxh[h], acc)
                    return c2

                lax.fori_loop(0, 8, grp, 0)
                pltpu.sync_copy(lg, lg_h.at[pl.ds(t * 128, 128)])

            return carry

        lax.fori_loop(0, n_iter, body, 0)
        res = jnp.full((16,), BIG, jnp.float32)
        for h in range(HEADS):
            s = jnp.max(mxh[h])
            res = jnp.where(iota == h, s, res)
        mxb[...] = res
        pltpu.sync_copy(mxb, gmp_h.at[w])

    return k_(q, k, e_l, dst2d, src2d)


def _load_gmax(gmp_h, gm, gv):
    pltpu.sync_copy(gmp_h, gm)
    g = gm[0]
    for r in range(1, NW):
        g = jnp.maximum(g, gm[r])
    gv[...] = g


def _sc_den_scatter(logits, gmp, dst2d, zeros16):
    nch = E // 512

    @functools.partial(
        pl.kernel,
        out_type=jax.ShapeDtypeStruct((2, N, 16), jnp.float32),
        mesh=_MESH,
        compiler_params=_SC_PARAMS,
        scratch_types=[
            pltpu.VMEM((512, 16), jnp.float32),
            pltpu.VMEM((512, 16), jnp.float32),
            pltpu.VMEM((4, 128), jnp.int32),
            pltpu.VMEM((NW, 16), jnp.float32),
            pltpu.VMEM((16,), jnp.float32),
            pltpu.VMEM_SHARED((N, 16), jnp.float32),
            pltpu.SemaphoreType.DMA,
        ],
    )
    def k_(lg_h, g_h, dst_h, z_h, den_h, lg, ex, dbuf, gm, gv, den_sp, sem):
        cid = lax.axis_index("c")
        sid = lax.axis_index("s")
        w = cid * 16 + sid
        pltpu.sync_copy(z_h.at[pl.ds(sid * ROWS_PER_TILE, ROWS_PER_TILE)],
                        den_sp.at[pl.ds(sid * ROWS_PER_TILE, ROWS_PER_TILE)])
        _load_gmax(g_h, gm, gv)
        plsc.subcore_barrier()
        gvec = gv[...]
        n_iter = (nch + NW - 1) // NW

        def body(i, carry):
            t = w + i * NW

            @pl.when(t < nch)
            def _():
                pltpu.sync_copy(lg_h.at[pl.ds(t * 512, 512)], lg)
                pltpu.sync_copy(dst_h.at[pl.ds(t * 4, 4)], dbuf)

                def inner(b, c2):
                    ex[b] = jnp.exp(lg[b] - gvec)
                    return c2

                lax.fori_loop(0, 512, inner, 0)
                for j in range(4):
                    pltpu.sync_copy(ex.at[pl.ds(j * 128, 128)],
                                    den_sp.at[dbuf.at[j]], add=True)

            return carry

        lax.fori_loop(0, n_iter, body, 0)
        plsc.subcore_barrier()
        pltpu.sync_copy(den_sp.at[pl.ds(sid * ROWS_PER_TILE, ROWS_PER_TILE)],
                        den_h.at[cid].at[pl.ds(sid * ROWS_PER_TILE,
                                               ROWS_PER_TILE)])

    return k_(logits, gmp, dst2d, zeros16)


def _sc_agg_scatter(logits, gmp, e_l, v, den0, den1, dst2d, src2d,
                    zeros128):
    nch = E // 128

    @functools.partial(
        pl.kernel,
        out_type=jax.ShapeDtypeStruct((2, N, HID), jnp.float32),
        mesh=_MESH,
        compiler_params=_SC_PARAMS,
        scratch_types=[
            pltpu.VMEM((128, 16), jnp.float32),
            pltpu.VMEM((128, HID), jnp.float32),
            pltpu.VMEM((128, HID), jnp.float32),
            pltpu.VMEM((128, 16), jnp.float32),
            pltpu.VMEM((128, 16), jnp.float32),
            pltpu.VMEM((1, 128), jnp.int32),
            pltpu.VMEM((1, 128), jnp.int32),
            pltpu.VMEM((NW, 16), jnp.float32),
            pltpu.VMEM((16,), jnp.float32),
            pltpu.VMEM_SHARED((N, HID), jnp.float32),
            pltpu.SemaphoreType.DMA,
        ],
    )
    def k_(lg_h, g_h, e_h, v_h, d0_h, d1_h, dst_h, src_h, z_h, agg_h,
           lg, ev, vs, d0, d1, dbuf, sbuf, gm, gv, agg_sp, sem):
        cid = lax.axis_index("c")
        sid = lax.axis_index("s")
        w = cid * 16 + sid
        pltpu.sync_copy(z_h.at[pl.ds(sid * ROWS_PER_TILE, ROWS_PER_TILE)],
                        agg_sp.at[pl.ds(sid * ROWS_PER_TILE, ROWS_PER_TILE)])
        _load_gmax(g_h, gm, gv)
        plsc.subcore_barrier()
        gvec = gv[...]
        n_iter = (nch + NW - 1) // NW

        def body(i, carry):
            t = w + i * NW

            @pl.when(t < nch)
            def _():
                pltpu.sync_copy(dst_h.at[pl.ds(t, 1)], dbuf)
                pltpu.sync_copy(src_h.at[pl.ds(t, 1)], sbuf)
                pltpu.sync_copy(lg_h.at[pl.ds(t * 128, 128)], lg)
                pltpu.sync_copy(e_h.at[pl.ds(t * 128, 128)], ev)
                descs = [
                    pltpu.async_copy(v_h.at[sbuf.at[0]], vs, sem),
                    pltpu.async_copy(d0_h.at[dbuf.at[0]], d0, sem),
                    pltpu.async_copy(d1_h.at[dbuf.at[0]], d1, sem),
                ]
                for d in descs:
                    d.wait()

                def inner(b, c2):
                    exv = jnp.exp(lg[b] - gvec)
                    denv = d0[b] + d1[b] + 1e-16
                    alv = exv / denv
                    for h in range(HEADS):
                        ev[b, pl.ds(h * 16, 16)] = (
                            vs[b, pl.ds(h * 16, 16)] + ev[b, pl.ds(h * 16, 16)]
                        ) * alv[h]
                    return c2

                lax.fori_loop(0, 128, inner, 0)
                pltpu.sync_copy(ev, agg_sp.at[dbuf.at[0]], add=True)

            return carry

        lax.fori_loop(0, n_iter, body, 0)
        plsc.subcore_barrier()
        pltpu.sync_copy(agg_sp.at[pl.ds(sid * ROWS_PER_TILE, ROWS_PER_TILE)],
                        agg_h.at[cid].at[pl.ds(sid * ROWS_PER_TILE,
                                               ROWS_PER_TILE)])

    return k_(logits, gmp, e_l, v, den0, den1, dst2d, src2d, zeros128)


def _tc_matmul_bias(x, w, b, blk):
    m, kdim = x.shape
    n = w.shape[1]
    grid = (m + blk - 1) // blk

    def body(x_r, w_r, b_r, o_r):
        o_r[...] = jnp.dot(x_r[...], w_r[...],
                           preferred_element_type=jnp.float32) + b_r[...]

    return pl.pallas_call(
        body,
        grid=(grid,),
        in_specs=[
            pl.BlockSpec((blk, kdim), lambda i: (i, 0)),
            pl.BlockSpec((kdim, n), lambda i: (0, 0)),
            pl.BlockSpec((1, n), lambda i: (0, 0)),
        ],
        out_specs=pl.BlockSpec((blk, n), lambda i: (i, 0)),
        out_shape=jax.ShapeDtypeStruct((m, n), jnp.float32),
    )(x, w, b)


def _tc_fold_edge_weights(W_edge, b_edge, We):
    ein = W_edge.shape[0]

    epad = ein + 8

    def body(we_r, wl_r, be_r, wc_r):
        wl = wl_r[0]
        wc = jnp.dot(we_r[...], wl, preferred_element_type=jnp.float32)
        bc = jnp.dot(be_r[...], wl, preferred_element_type=jnp.float32)
        wc_r[0] = jnp.concatenate(
            [wc, bc, jnp.zeros((epad - ein - 1, HID), jnp.float32)], axis=0)

    return pl.pallas_call(
        body,
        grid=(L,),
        in_specs=[
            pl.BlockSpec((ein, HID), lambda i: (0, 0)),
            pl.BlockSpec((1, HID, HID), lambda i: (i, 0, 0)),
            pl.BlockSpec((1, HID), lambda i: (0, 0)),
        ],
        out_specs=pl.BlockSpec((1, epad, HID), lambda i: (i, 0, 0)),
        out_shape=jax.ShapeDtypeStruct((L, epad, HID), jnp.float32),
    )(W_edge, We, b_edge[None, :])


def _tc_proj4(x, wq, wk, wv, wskip, bskip):
    blk = 1024
    grid = (N + blk - 1) // blk

    def body(x_r, wq_r, wk_r, wv_r, ws_r, bs_r, q_r, k_r, v_r, xr_r):
        xb = x_r[...]
        q_r[...] = jnp.dot(xb, wq_r[...], preferred_element_type=jnp.float32)
        k_r[...] = jnp.dot(xb, wk_r[...], preferred_element_type=jnp.float32)
        v_r[...] = jnp.dot(xb, wv_r[...], preferred_element_type=jnp.float32)
        xr_r[...] = jnp.dot(xb, ws_r[...],
                            preferred_element_type=jnp.float32) + bs_r[...]

    o = jax.ShapeDtypeStruct((N, HID), jnp.float32)
    wspec = pl.BlockSpec((HID, HID), lambda i: (0, 0))
    return pl.pallas_call(
        body,
        grid=(grid,),
        in_specs=[pl.BlockSpec((blk, HID), lambda i: (i, 0)),
                  wspec, wspec, wspec, wspec,
                  pl.BlockSpec((1, HID), lambda i: (0, 0))],
        out_specs=[pl.BlockSpec((blk, HID), lambda i: (i, 0))] * 4,
        out_shape=[o, o, o, o],
    )(x, wq, wk, wv, wskip, bskip)


def _tc_post(x, agg0, agg1, xr, wba, wbx, g1, b1, wf1, bf1, wf2, bf2, g2, b2):
    blk = 1024
    grid = (N + blk - 1) // blk

    def ln(y, g, b):
        m = jnp.mean(y, axis=-1, keepdims=True)
        v = jnp.mean((y - m) ** 2, axis=-1, keepdims=True)
        return g * (y - m) / jnp.sqrt(v + 1e-5) + b

    def body(x_r, a0_r, a1_r, xr_r, wba_r, wbx_r, g1_r, b1_r,
             wf1_r, bf1_r, wf2_r, bf2_r, g2_r, b2_r, o_r):
        agg = a0_r[...] + a1_r[...]
        xrb = xr_r[...]
        bl = (jnp.dot(agg, wba_r[...], preferred_element_type=jnp.float32)
              + jnp.dot(xrb, wbx_r[...], preferred_element_type=jnp.float32))
        beta = jax.nn.sigmoid(bl)
        h = beta * xrb + (1.0 - beta) * agg
        y = ln(x_r[...] + h, g1_r[...], b1_r[...])
        h2 = jnp.dot(
            jax.nn.gelu(jnp.dot(y, wf1_r[...],
                                preferred_element_type=jnp.float32)
                        + bf1_r[...]),
            wf2_r[...], preferred_element_type=jnp.float32) + bf2_r[...]
        o_r[...] = ln(y + h2, g2_r[...], b2_r[...])

    nblk = pl.BlockSpec((blk, HID), lambda i: (i, 0))
    row = pl.BlockSpec((1, HID), lambda i: (0, 0))
    return pl.pallas_call(
        body,
        grid=(grid,),
        in_specs=[
            nblk, nblk, nblk, nblk,
            pl.BlockSpec((HID, 1), lambda i: (0, 0)),
            pl.BlockSpec((HID, 1), lambda i: (0, 0)),
            row, row,
            pl.BlockSpec((HID, 4 * HID), lambda i: (0, 0)),
            pl.BlockSpec((1, 4 * HID), lambda i: (0, 0)),
            pl.BlockSpec((4 * HID, HID), lambda i: (0, 0)),
            row, row, row,
        ],
        out_specs=nblk,
        out_shape=jax.ShapeDtypeStruct((N, HID), jnp.float32),
    )(x, agg0, agg1, xr, wba, wbx, g1, b1, wf1, bf1, wf2, bf2, g2, b2)


def kernel(x_cont, node_cat, lookahead_cat, package_postal, edge_index,
           edge_cont, edge_cat, node_tables, lookahead_tables, edge_tables,
           postal_table, W_node, b_node, W_edge, b_edge, Wq, Wk, Wv, We,
           Wskip, bskip, Wbeta, ln1_g, ln1_b, Wf1, bf1, Wf2, bf2,
           ln2_g, ln2_b):
    i32 = jnp.int32
    f32 = jnp.float32

    stacked = jnp.concatenate([
        node_tables.reshape(-1, EMBED),
        lookahead_tables.reshape(-1, EMBED),
        postal_table,
        edge_tables.reshape(-1, EMBED),
    ], axis=0)
    offn = (jnp.arange(9, dtype=i32) * VOCAB)[None, :]
    offl = ((9 + jnp.arange(7, dtype=i32)) * VOCAB)[None, :]
    offe = ((17 + jnp.arange(9, dtype=i32)) * VOCAB)[None, :]
    idx_n = jnp.concatenate([
        node_cat.astype(i32) + offn,
        lookahead_cat.astype(i32) + offl,
        package_postal.astype(i32) + 16 * VOCAB,
    ], axis=1).reshape(-1)
    idx_e = (edge_cat.astype(i32) + offe).reshape(-1)
    idx_all = jnp.concatenate([idx_n, idx_e])
    total = idx_all.shape[0]
    n_chunks = (total + 1023) // 1024
    pad = n_chunks * 1024 - total
    idx_all = jnp.concatenate([idx_all, jnp.zeros((pad,), i32)])
    idx2d = idx_all.reshape(n_chunks * 8, 128)

    gathered = _sc_embed_gather(stacked, idx2d, n_chunks)
    node_emb = gathered[:180000].reshape(N, 18 * EMBED)
    edge_emb = gathered[180000:180000 + 9 * E].reshape(E, 9 * EMBED)

    x_in = jnp.concatenate([x_cont, node_emb], axis=1)
    x = _tc_matmul_bias(x_in, W_node, b_node[None, :], 1024)
    ef_in = jnp.concatenate([edge_cont, edge_emb], axis=1)
    folded = _tc_fold_edge_weights(W_edge, b_edge, We)

    src2d = edge_index[0].astype(i32).reshape(E // 128, 128)
    dst2d = edge_index[1].astype(i32).reshape(E // 128, 128)

    z16 = jnp.zeros((N, 16), f32)
    z128 = jnp.zeros((N, HID), f32)

    for l in range(L):
        q, k, v, xr = _tc_proj4(x, Wq[l], Wk[l], Wv[l], Wskip[l],
                                bskip[l][None, :])
        e_l = _tc_matmul_bias(ef_in, folded[l, :152], folded[l, 152:153],
                              2048)
        logits, gmp = _sc_e1(q, k, e_l, dst2d, src2d)
        den = _sc_den_scatter(logits, gmp, dst2d, z16)
        agg = _sc_agg_scatter(logits, gmp, e_l, v, den[0], den[1],
                              dst2d, src2d, z128)
        wb = Wbeta[l]
        wba = wb[:HID] + wb[2 * HID:]
        wbx = wb[HID:2 * HID] - wb[2 * HID:]
        x = _tc_post(x, agg[0], agg[1], xr, wba, wbx,
                     ln1_g[l][None, :], ln1_b[l][None, :],
                     Wf1[l], bf1[l][None, :], Wf2[l], bf2[l][None, :],
                     ln2_g[l][None, :], ln2_b[l][None, :])
    return x


# speedup vs baseline: 1.4669x; 1.4669x over previous
import functools

import jax
import jax.numpy as jnp
import numpy as np
from jax import lax
from jax.experimental import pallas as pl
from jax.experimental.pallas import tpu as pltpu
from jax.experimental.pallas import tpu_sc as plsc

N = 10000
E = 320000
EMBED = 16
HID = 128
HEADS = 8
DH = 16
VOCAB = 20000
L = 2
NEG = -1e30
BIG = 1e30

_MESH = plsc.VectorSubcoreMesh(core_axis_name="c", subcore_axis_name="s")
_SC_PARAMS = pltpu.CompilerParams(use_tc_tiling_on_sc=False)
_SC_PARAMS_NL = pltpu.CompilerParams(use_tc_tiling_on_sc=False,
                                     needs_layout_passes=False)
NW = 32
ROWS_PER_TILE = N // 16


def _wid():
    return lax.axis_index("c") * 16 + lax.axis_index("s")


def _sc_embed_gather(table, idx2d, n_chunks):
    @functools.partial(
        pl.kernel,
        out_type=jax.ShapeDtypeStruct((n_chunks * 1024, 16), jnp.float32),
        mesh=_MESH,
        compiler_params=_SC_PARAMS,
        scratch_types=[
            pltpu.VMEM((8, 128), jnp.int32),
            pltpu.VMEM((1024, 16), jnp.float32),
            pltpu.SemaphoreType.DMA,
        ],
    )
    def k(table_h, idx_h, out_h, idx_v, rows_v, sem):
        w = _wid()
        n_iter = (n_chunks + NW - 1) // NW

        def body(i, carry):
            t = w + i * NW

            @pl.when(t < n_chunks)
            def _():
                pltpu.sync_copy(idx_h.at[pl.ds(t * 8, 8)], idx_v)
                descs = [
                    pltpu.async_copy(
                        table_h.at[idx_v.at[j]],
                        rows_v.at[pl.ds(j * 128, 128)],
                        sem,
                    )
                    for j in range(8)
                ]
                for d in descs:
                    d.wait()
                pltpu.sync_copy(rows_v, out_h.at[pl.ds(t * 1024, 1024)])

            return carry

        lax.fori_loop(0, n_iter, body, 0)

    return k(table, idx2d)


def _sc_e1(q, k, e_l, dst2d, src2d):
    nch = E // 128

    @functools.partial(
        pl.kernel,
        out_type=(
            jax.ShapeDtypeStruct((E, 16), jnp.float32),
            jax.ShapeDtypeStruct((NW, 16), jnp.float32),
        ),
        mesh=_MESH,
        compiler_params=_SC_PARAMS_NL,
        scratch_types=[
            pltpu.VMEM((1, 128), jnp.int32),
            pltpu.VMEM((1, 128), jnp.int32),
            pltpu.VMEM((128, HID), jnp.float32),
            pltpu.VMEM((128, HID), jnp.float32),
            pltpu.VMEM((128, HID), jnp.float32),
            pltpu.VMEM((128, 16), jnp.float32),
            pltpu.VMEM((8, 16), jnp.float32),
            pltpu.VMEM((16,), jnp.float32),
            pltpu.SemaphoreType.DMA,
        ],
    )
    def k_(q_h, k_h, e_h, dst_h, src_h, lg_h, gmp_h,
           dbuf, sbuf, qb, kb, eb, lg, mxh, mxb, sem):
        w = _wid()
        n_iter = (nch + NW - 1) // NW
        iota = lax.iota(jnp.int32, 16)
        for h in range(HEADS):
            mxh[h] = jnp.full((16,), NEG, jnp.float32)

        def body(i, carry):
            t = w + i * NW

            @pl.when(t < nch)
            def _():
                pltpu.sync_copy(dst_h.at[pl.ds(t, 1)], dbuf)
                pltpu.sync_copy(src_h.at[pl.ds(t, 1)], sbuf)
                descs = [
                    pltpu.async_copy(q_h.at[dbuf.at[0]], qb, sem),
                    pltpu.async_copy(k_h.at[sbuf.at[0]], kb, sem),
                    pltpu.async_copy(e_h.at[pl.ds(t * 128, 128)], eb, sem),
                ]
                for d in descs:
                    d.wait()

                def grp(g, c2):
                    ri = g * 16 + iota
                    for h in range(HEADS):
                        acc = jnp.zeros((16,), jnp.float32)
                        for j in range(DH):
                            cs = h * DH + ((iota + j) & 15)
                            qv = plsc.load_gather(qb, [ri, cs])
                            kv = plsc.load_gather(kb, [ri, cs])
                            ev = plsc.load_gather(eb, [ri, cs])
                            acc = acc + qv * (kv + ev)
                        acc = acc * 0.25
                        plsc.store_scatter(
                            lg, [ri, jnp.full((16,), h, jnp.int32)], acc)
                        mxh[h] = jnp.maximum(mxh[h], acc)
                    return c2

                lax.fori_loop(0, 8, grp, 0)
                pltpu.sync_copy(lg, lg_h.at[pl.ds(t * 128, 128)])

            return carry

        lax.fori_loop(0, n_iter, body, 0)
        res = jnp.full((16,), BIG, jnp.float32)
        for h in range(HEADS):
            s = jnp.max(mxh[h])
            res = jnp.where(iota == h, s, res)
        mxb[...] = res
        pltpu.sync_copy(mxb, gmp_h.at[w])

    return k_(q, k, e_l, dst2d, src2d)


def _load_gmax(gmp_h, gm, gv):
    pltpu.sync_copy(gmp_h, gm)
    g = gm[0]
    for r in range(1, NW):
        g = jnp.maximum(g, gm[r])
    gv[...] = g


def _sc_den_scatter(logits, gmp, dst2d, zeros16):
    nch = E // 512

    @functools.partial(
        pl.kernel,
        out_type=jax.ShapeDtypeStruct((2, N, 16), jnp.float32),
        mesh=_MESH,
        compiler_params=_SC_PARAMS,
        scratch_types=[
            pltpu.VMEM((512, 16), jnp.float32),
            pltpu.VMEM((512, 16), jnp.float32),
            pltpu.VMEM((4, 128), jnp.int32),
            pltpu.VMEM((NW, 16), jnp.float32),
            pltpu.VMEM((16,), jnp.float32),
            pltpu.VMEM_SHARED((N, 16), jnp.float32),
            pltpu.SemaphoreType.DMA,
        ],
    )
    def k_(lg_h, g_h, dst_h, z_h, den_h, lg, ex, dbuf, gm, gv, den_sp, sem):
        cid = lax.axis_index("c")
        sid = lax.axis_index("s")
        w = cid * 16 + sid
        pltpu.sync_copy(z_h.at[pl.ds(sid * ROWS_PER_TILE, ROWS_PER_TILE)],
                        den_sp.at[pl.ds(sid * ROWS_PER_TILE, ROWS_PER_TILE)])
        _load_gmax(g_h, gm, gv)
        plsc.subcore_barrier()
        gvec = gv[...]
        n_iter = (nch + NW - 1) // NW

        def body(i, carry):
            t = w + i * NW

            @pl.when(t < nch)
            def _():
                pltpu.sync_copy(lg_h.at[pl.ds(t * 512, 512)], lg)
                pltpu.sync_copy(dst_h.at[pl.ds(t * 4, 4)], dbuf)

                def inner(b, c2):
                    ex[b] = jnp.exp(lg[b] - gvec)
                    return c2

                lax.fori_loop(0, 512, inner, 0)
                for j in range(4):
                    pltpu.sync_copy(ex.at[pl.ds(j * 128, 128)],
                                    den_sp.at[dbuf.at[j]], add=True)

            return carry

        lax.fori_loop(0, n_iter, body, 0)
        plsc.subcore_barrier()
        pltpu.sync_copy(den_sp.at[pl.ds(sid * ROWS_PER_TILE, ROWS_PER_TILE)],
                        den_h.at[cid].at[pl.ds(sid * ROWS_PER_TILE,
                                               ROWS_PER_TILE)])

    return k_(logits, gmp, dst2d, zeros16)


def _sc_agg_scatter(logits, gmp, e_l, v, den0, den1, dst2d, src2d,
                    zeros128):
    nch = E // 128

    @functools.partial(
        pl.kernel,
        out_type=jax.ShapeDtypeStruct((2, N, HID), jnp.float32),
        mesh=_MESH,
        compiler_params=_SC_PARAMS,
        scratch_types=[
            pltpu.VMEM((128, 16), jnp.float32),
            pltpu.VMEM((128, HID), jnp.float32),
            pltpu.VMEM((128, HID), jnp.float32),
            pltpu.VMEM((128, 16), jnp.float32),
            pltpu.VMEM((128, 16), jnp.float32),
            pltpu.VMEM((1, 128), jnp.int32),
            pltpu.VMEM((1, 128), jnp.int32),
            pltpu.VMEM((NW, 16), jnp.float32),
            pltpu.VMEM((16,), jnp.float32),
            pltpu.VMEM_SHARED((N, HID), jnp.float32),
            pltpu.SemaphoreType.DMA,
        ],
    )
    def k_(lg_h, g_h, e_h, v_h, d0_h, d1_h, dst_h, src_h, z_h, agg_h,
           lg, ev, vs, d0, d1, dbuf, sbuf, gm, gv, agg_sp, sem):
        cid = lax.axis_index("c")
        sid = lax.axis_index("s")
        w = cid * 16 + sid
        pltpu.sync_copy(z_h.at[pl.ds(sid * ROWS_PER_TILE, ROWS_PER_TILE)],
                        agg_sp.at[pl.ds(sid * ROWS_PER_TILE, ROWS_PER_TILE)])
        _load_gmax(g_h, gm, gv)
        plsc.subcore_barrier()
        gvec = gv[...]
        n_iter = (nch + NW - 1) // NW

        def body(i, carry):
            t = w + i * NW

            @pl.when(t < nch)
            def _():
                pltpu.sync_copy(dst_h.at[pl.ds(t, 1)], dbuf)
                pltpu.sync_copy(src_h.at[pl.ds(t, 1)], sbuf)
                pltpu.sync_copy(lg_h.at[pl.ds(t * 128, 128)], lg)
                pltpu.sync_copy(e_h.at[pl.ds(t * 128, 128)], ev)
                descs = [
                    pltpu.async_copy(v_h.at[sbuf.at[0]], vs, sem),
                    pltpu.async_copy(d0_h.at[dbuf.at[0]], d0, sem),
                    pltpu.async_copy(d1_h.at[dbuf.at[0]], d1, sem),
                ]
                for d in descs:
                    d.wait()

                def inner(b, c2):
                    exv = jnp.exp(lg[b] - gvec)
                    denv = d0[b] + d1[b] + 1e-16
                    alv = exv / denv
                    for h in range(HEADS):
                        ev[b, pl.ds(h * 16, 16)] = (
                            vs[b, pl.ds(h * 16, 16)] + ev[b, pl.ds(h * 16, 16)]
                        ) * alv[h]
                    return c2

                lax.fori_loop(0, 128, inner, 0)
                pltpu.sync_copy(ev, agg_sp.at[dbuf.at[0]], add=True)

            return carry

        lax.fori_loop(0, n_iter, body, 0)
        plsc.subcore_barrier()
        pltpu.sync_copy(agg_sp.at[pl.ds(sid * ROWS_PER_TILE, ROWS_PER_TILE)],
                        agg_h.at[cid].at[pl.ds(sid * ROWS_PER_TILE,
                                               ROWS_PER_TILE)])

    return k_(logits, gmp, e_l, v, den0, den1, dst2d, src2d, zeros128)


def _tc_matmul_bias(x, w, b, blk):
    m, kdim = x.shape
    n = w.shape[1]
    grid = (m + blk - 1) // blk

    def body(x_r, w_r, b_r, o_r):
        o_r[...] = jnp.dot(x_r[...], w_r[...],
                           preferred_element_type=jnp.float32) + b_r[...]

    return pl.pallas_call(
        body,
        grid=(grid,),
        in_specs=[
            pl.BlockSpec((blk, kdim), lambda i: (i, 0)),
            pl.BlockSpec((kdim, n), lambda i: (0, 0)),
            pl.BlockSpec((1, n), lambda i: (0, 0)),
        ],
        out_specs=pl.BlockSpec((blk, n), lambda i: (i, 0)),
        out_shape=jax.ShapeDtypeStruct((m, n), jnp.float32),
    )(x, w, b)


def _tc_fold_edge_weights(W_edge, b_edge, We):
    ein = W_edge.shape[0]

    epad = ein + 8

    def body(we_r, wl_r, be_r, wc_r):
        wl = wl_r[0]
        wc = jnp.dot(we_r[...], wl, preferred_element_type=jnp.float32)
        bc = jnp.dot(be_r[...], wl, preferred_element_type=jnp.float32)
        wc_r[0] = jnp.concatenate(
            [wc, bc, jnp.zeros((epad - ein - 1, HID), jnp.float32)], axis=0)

    return pl.pallas_call(
        body,
        grid=(L,),
        in_specs=[
            pl.BlockSpec((ein, HID), lambda i: (0, 0)),
            pl.BlockSpec((1, HID, HID), lambda i: (i, 0, 0)),
            pl.BlockSpec((1, HID), lambda i: (0, 0)),
        ],
        out_specs=pl.BlockSpec((1, epad, HID), lambda i: (i, 0, 0)),
        out_shape=jax.ShapeDtypeStruct((L, epad, HID), jnp.float32),
    )(W_edge, We, b_edge[None, :])


def _tc_proj4(x, wq, wk, wv, wskip, bskip):
    blk = 1024
    grid = (N + blk - 1) // blk

    def body(x_r, wq_r, wk_r, wv_r, ws_r, bs_r, q_r, k_r, v_r, xr_r):
        xb = x_r[...]
        q_r[...] = jnp.dot(xb, wq_r[...], preferred_element_type=jnp.float32)
        k_r[...] = jnp.dot(xb, wk_r[...], preferred_element_type=jnp.float32)
        v_r[...] = jnp.dot(xb, wv_r[...], preferred_element_type=jnp.float32)
        xr_r[...] = jnp.dot(xb, ws_r[...],
                            preferred_element_type=jnp.float32) + bs_r[...]

    o = jax.ShapeDtypeStruct((N, HID), jnp.float32)
    wspec = pl.BlockSpec((HID, HID), lambda i: (0, 0))
    return pl.pallas_call(
        body,
        grid=(grid,),
        in_specs=[pl.BlockSpec((blk, HID), lambda i: (i, 0)),
                  wspec, wspec, wspec, wspec,
                  pl.BlockSpec((1, HID), lambda i: (0, 0))],
        out_specs=[pl.BlockSpec((blk, HID), lambda i: (i, 0))] * 4,
        out_shape=[o, o, o, o],
    )(x, wq, wk, wv, wskip, bskip)


def _tc_post(x, agg0, agg1, xr, wba, wbx, g1, b1, wf1, bf1, wf2, bf2, g2, b2):
    blk = 1024
    grid = (N + blk - 1) // blk

    def ln(y, g, b):
        m = jnp.mean(y, axis=-1, keepdims=True)
        v = jnp.mean((y - m) ** 2, axis=-1, keepdims=True)
        return g * (y - m) / jnp.sqrt(v + 1e-5) + b

    def body(x_r, a0_r, a1_r, xr_r, wba_r, wbx_r, g1_r, b1_r,
             wf1_r, bf1_r, wf2_r, bf2_r, g2_r, b2_r, o_r):
        agg = a0_r[...] + a1_r[...]
        xrb = xr_r[...]
        bl = (jnp.dot(agg, wba_r[...], preferred_element_type=jnp.float32)
              + jnp.dot(xrb, wbx_r[...], preferred_element_type=jnp.float32))
        beta = jax.nn.sigmoid(bl)
        h = beta * xrb + (1.0 - beta) * agg
        y = ln(x_r[...] + h, g1_r[...], b1_r[...])
        h2 = jnp.dot(
            jax.nn.gelu(jnp.dot(y, wf1_r[...],
                                preferred_element_type=jnp.float32)
                        + bf1_r[...]),
            wf2_r[...], preferred_element_type=jnp.float32) + bf2_r[...]
        o_r[...] = ln(y + h2, g2_r[...], b2_r[...])

    nblk = pl.BlockSpec((blk, HID), lambda i: (i, 0))
    row = pl.BlockSpec((1, HID), lambda i: (0, 0))
    return pl.pallas_call(
        body,
        grid=(grid,),
        in_specs=[
            nblk, nblk, nblk, nblk,
            pl.BlockSpec((HID, 1), lambda i: (0, 0)),
            pl.BlockSpec((HID, 1), lambda i: (0, 0)),
            row, row,
            pl.BlockSpec((HID, 4 * HID), lambda i: (0, 0)),
            pl.BlockSpec((1, 4 * HID), lambda i: (0, 0)),
            pl.BlockSpec((4 * HID, HID), lambda i: (0, 0)),
            row, row, row,
        ],
        out_specs=nblk,
        out_shape=jax.ShapeDtypeStruct((N, HID), jnp.float32),
    )(x, agg0, agg1, xr, wba, wbx, g1, b1, wf1, bf1, wf2, bf2, g2, b2)


def kernel(x_cont, node_cat, lookahead_cat, package_postal, edge_index,
           edge_cont, edge_cat, node_tables, lookahead_tables, edge_tables,
           postal_table, W_node, b_node, W_edge, b_edge, Wq, Wk, Wv, We,
           Wskip, bskip, Wbeta, ln1_g, ln1_b, Wf1, bf1, Wf2, bf2,
           ln2_g, ln2_b):
    i32 = jnp.int32
    f32 = jnp.float32

    stacked = jnp.concatenate([
        node_tables.reshape(-1, EMBED),
        lookahead_tables.reshape(-1, EMBED),
        postal_table,
        edge_tables.reshape(-1, EMBED),
    ], axis=0)
    offn = (jnp.arange(9, dtype=i32) * VOCAB)[None, :]
    offl = ((9 + jnp.arange(7, dtype=i32)) * VOCAB)[None, :]
    offe = ((17 + jnp.arange(9, dtype=i32)) * VOCAB)[None, :]
    idx_n = jnp.concatenate([
        node_cat.astype(i32) + offn,
        lookahead_cat.astype(i32) + offl,
        package_postal.astype(i32) + 16 * VOCAB,
    ], axis=1).reshape(-1)
    idx_e = (edge_cat.astype(i32) + offe).reshape(-1)
    idx_all = jnp.concatenate([idx_n, idx_e])
    total = idx_all.shape[0]
    n_chunks = (total + 1023) // 1024
    pad = n_chunks * 1024 - total
    idx_all = jnp.concatenate([idx_all, jnp.zeros((pad,), i32)])
    idx2d = idx_all.reshape(n_chunks * 8, 128)

    gathered = _sc_embed_gather(stacked, idx2d, n_chunks)
    node_emb = gathered[:180000].reshape(N, 18 * EMBED)
    edge_emb = gathered[180000:180000 + 9 * E].reshape(E, 9 * EMBED)

    x_in = jnp.concatenate([x_cont, node_emb], axis=1)
    x = _tc_matmul_bias(x_in, W_node, b_node[None, :], 1024)
    ef_in = jnp.concatenate([edge_cont, edge_emb], axis=1)
    folded = _tc_fold_edge_weights(W_edge, b_edge, We)

    src2d = edge_index[0].astype(i32).reshape(E // 128, 128)
    dst2d = edge_index[1].astype(i32).reshape(E // 128, 128)

    z16 = jnp.zeros((N, 16), f32)
    z128 = jnp.zeros((N, HID), f32)

    for l in range(L):
        q, k, v, xr = _tc_proj4(x, Wq[l], Wk[l], Wv[l], Wskip[l],
                                bskip[l][None, :])
        e_l = _tc_matmul_bias(ef_in, folded[l, :152], folded[l, 152:153],
                              2048)
        logits, gmp = _sc_e1(q, k, e_l, dst2d, src2d)
        den = _sc_den_scatter(logits, gmp, dst2d, z16)
        agg = _sc_agg_scatter(logits, gmp, e_l, v, den[0], den[1],
                              dst2d, src2d, z128)
        wb = Wbeta[l]
        wba = wb[:HID] + wb[2 * HID:]
        wbx = wb[HID:2 * HID] - wb[2 * HID:]
        x = _tc_post(x, agg[0], agg[1], xr, wba, wbx,
                     ln1_g[l][None, :], ln1_b[l][None, :],
                     Wf1[l], bf1[l][None, :], Wf2[l], bf2[l][None, :],
                     ln2_g[l][None, :], ln2_b[l][None, :])
    return x


# speedup vs baseline: 1.6365x; 1.1156x over previous
import functools

import jax
import jax.numpy as jnp
import numpy as np
from jax import lax
from jax.experimental import pallas as pl
from jax.experimental.pallas import tpu as pltpu
from jax.experimental.pallas import tpu_sc as plsc

N = 10000
E = 320000
EMBED = 16
HID = 128
HEADS = 8
DH = 16
VOCAB = 20000
L = 2
NEG = -1e30
BIG = 1e30

_MESH = plsc.VectorSubcoreMesh(core_axis_name="c", subcore_axis_name="s")
_SC_PARAMS = pltpu.CompilerParams(use_tc_tiling_on_sc=False)
_SC_PARAMS_NL = pltpu.CompilerParams(use_tc_tiling_on_sc=False,
                                     needs_layout_passes=False)
NW = 32
ROWS_PER_TILE = N // 16


def _wid():
    return lax.axis_index("c") * 16 + lax.axis_index("s")


def _sc_embed_gather(table, idx2d, n_chunks):
    @functools.partial(
        pl.kernel,
        out_type=jax.ShapeDtypeStruct((n_chunks * 1024, 16), jnp.float32),
        mesh=_MESH,
        compiler_params=_SC_PARAMS,
        scratch_types=[
            pltpu.VMEM((8, 128), jnp.int32),
            pltpu.VMEM((1024, 16), jnp.float32),
            pltpu.SemaphoreType.DMA,
        ],
    )
    def k(table_h, idx_h, out_h, idx_v, rows_v, sem):
        w = _wid()
        n_iter = (n_chunks + NW - 1) // NW

        def body(i, carry):
            t = w + i * NW

            @pl.when(t < n_chunks)
            def _():
                pltpu.sync_copy(idx_h.at[pl.ds(t * 8, 8)], idx_v)
                descs = [
                    pltpu.async_copy(
                        table_h.at[idx_v.at[j]],
                        rows_v.at[pl.ds(j * 128, 128)],
                        sem,
                    )
                    for j in range(8)
                ]
                for d in descs:
                    d.wait()
                pltpu.sync_copy(rows_v, out_h.at[pl.ds(t * 1024, 1024)])

            return carry

        lax.fori_loop(0, n_iter, body, 0)

    return k(table, idx2d)


def _sc_e1(q, k, e_l, dst2d, src2d):
    nch = E // 128

    @functools.partial(
        pl.kernel,
        out_type=(
            jax.ShapeDtypeStruct((E, 16), jnp.float32),
            jax.ShapeDtypeStruct((NW, 16), jnp.float32),
        ),
        mesh=_MESH,
        compiler_params=_SC_PARAMS_NL,
        scratch_types=[
            pltpu.VMEM((1, 128), jnp.int32),
            pltpu.VMEM((1, 128), jnp.int32),
            pltpu.VMEM((1, 128), jnp.int32),
            pltpu.VMEM((1, 128), jnp.int32),
            pltpu.VMEM((128, HID), jnp.float32),
            pltpu.VMEM((128, HID), jnp.float32),
            pltpu.VMEM((128, HID), jnp.float32),
            pltpu.VMEM((128, HID), jnp.float32),
            pltpu.VMEM((128, HID), jnp.float32),
            pltpu.VMEM((128, HID), jnp.float32),
            pltpu.VMEM((128, 16), jnp.float32),
            pltpu.VMEM((8, 16), jnp.float32),
            pltpu.VMEM((16,), jnp.float32),
            pltpu.SemaphoreType.DMA,
            pltpu.SemaphoreType.DMA,
        ],
    )
    def k_(q_h, k_h, e_h, dst_h, src_h, lg_h, gmp_h,
           db0, db1, sb0, sb1, qb0, qb1, kb0, kb1, eb0, eb1,
           lg, mxh, mxb, sm0, sm1):
        dbuf, sbuf = (db0, db1), (sb0, sb1)
        qb, kb, eb = (qb0, qb1), (kb0, kb1), (eb0, eb1)
        sems = (sm0, sm1)
        w = _wid()
        n_iter = (nch + NW - 1) // NW
        iota = lax.iota(jnp.int32, 16)
        for h in range(HEADS):
            mxh[h] = jnp.full((16,), NEG, jnp.float32)

        def issue(t, p):
            pltpu.sync_copy(dst_h.at[pl.ds(t, 1)], dbuf[p])
            pltpu.sync_copy(src_h.at[pl.ds(t, 1)], sbuf[p])
            pltpu.async_copy(q_h.at[dbuf[p].at[0]], qb[p], sems[p])
            pltpu.async_copy(k_h.at[sbuf[p].at[0]], kb[p], sems[p])
            pltpu.async_copy(e_h.at[pl.ds(t * 128, 128)], eb[p], sems[p])

        def drain(t, p):
            pltpu.make_async_copy(
                q_h.at[dbuf[p].at[0]], qb[p], sems[p]).wait()
            pltpu.make_async_copy(
                k_h.at[sbuf[p].at[0]], kb[p], sems[p]).wait()
            pltpu.make_async_copy(
                e_h.at[pl.ds(t * 128, 128)], eb[p], sems[p]).wait()

        @pl.when(w < nch)
        def _():
            issue(w, 0)

        def compute(qbp, kbp, ebp):

                def grp(g, c2):
                    ri = g * 16 + iota
                    for h in range(HEADS):
                        acc = jnp.zeros((16,), jnp.float32)
                        for j in range(DH):
                            cs = h * DH + ((iota + j) & 15)
                            qv = plsc.load_gather(qbp, [ri, cs])
                            kv = plsc.load_gather(kbp, [ri, cs])
                            ev = plsc.load_gather(ebp, [ri, cs])
                            acc = acc + qv * (kv + ev)
                        acc = acc * 0.25
                        plsc.store_scatter(
                            lg, [ri, jnp.full((16,), h, jnp.int32)], acc)
                        mxh[h] = jnp.maximum(mxh[h], acc)
                    return c2

                lax.fori_loop(0, 8, grp, 0)

        def body(i, carry):
            t = w + i * NW

            @pl.when(t < nch)
            def _():
                for p in range(2):

                    @pl.when(i % 2 == p)
                    def _():
                        drain(t, p)

                        @pl.when(t + NW < nch)
                        def _():
                            issue(t + NW, 1 - p)

                        compute(qb[p], kb[p], eb[p])

                pltpu.sync_copy(lg, lg_h.at[pl.ds(t * 128, 128)])

            return carry

        lax.fori_loop(0, n_iter, body, 0)
        res = jnp.full((16,), BIG, jnp.float32)
        for h in range(HEADS):
            s = jnp.max(mxh[h])
            res = jnp.where(iota == h, s, res)
        mxb[...] = res
        pltpu.sync_copy(mxb, gmp_h.at[w])

    return k_(q, k, e_l, dst2d, src2d)


def _load_gmax(gmp_h, gm, gv):
    pltpu.sync_copy(gmp_h, gm)
    g = gm[0]
    for r in range(1, NW):
        g = jnp.maximum(g, gm[r])
    gv[...] = g


def _sc_den_scatter(logits, gmp, dst2d, zeros16):
    nch = E // 512

    @functools.partial(
        pl.kernel,
        out_type=jax.ShapeDtypeStruct((2, N, 16), jnp.float32),
        mesh=_MESH,
        compiler_params=_SC_PARAMS,
        scratch_types=[
            pltpu.VMEM((512, 16), jnp.float32),
            pltpu.VMEM((512, 16), jnp.float32),
            pltpu.VMEM((4, 128), jnp.int32),
            pltpu.VMEM((NW, 16), jnp.float32),
            pltpu.VMEM((16,), jnp.float32),
            pltpu.VMEM_SHARED((N, 16), jnp.float32),
            pltpu.SemaphoreType.DMA,
        ],
    )
    def k_(lg_h, g_h, dst_h, z_h, den_h, lg, ex, dbuf, gm, gv, den_sp, sem):
        cid = lax.axis_index("c")
        sid = lax.axis_index("s")
        w = cid * 16 + sid
        pltpu.sync_copy(z_h.at[pl.ds(sid * ROWS_PER_TILE, ROWS_PER_TILE)],
                        den_sp.at[pl.ds(sid * ROWS_PER_TILE, ROWS_PER_TILE)])
        _load_gmax(g_h, gm, gv)
        plsc.subcore_barrier()
        gvec = gv[...]
        n_iter = (nch + NW - 1) // NW

        def body(i, carry):
            t = w + i * NW

            @pl.when(t < nch)
            def _():
                pltpu.sync_copy(lg_h.at[pl.ds(t * 512, 512)], lg)
                pltpu.sync_copy(dst_h.at[pl.ds(t * 4, 4)], dbuf)

                def inner(b, c2):
                    ex[b] = jnp.exp(lg[b] - gvec)
                    return c2

                lax.fori_loop(0, 512, inner, 0)
                for j in range(4):
                    pltpu.sync_copy(ex.at[pl.ds(j * 128, 128)],
                                    den_sp.at[dbuf.at[j]], add=True)

            return carry

        lax.fori_loop(0, n_iter, body, 0)
        plsc.subcore_barrier()
        pltpu.sync_copy(den_sp.at[pl.ds(sid * ROWS_PER_TILE, ROWS_PER_TILE)],
                        den_h.at[cid].at[pl.ds(sid * ROWS_PER_TILE,
                                               ROWS_PER_TILE)])

    return k_(logits, gmp, dst2d, zeros16)


def _sc_agg_scatter(logits, gmp, e_l, v, den0, den1, dst64, src64,
                    zeros128):
    blk = 64
    nch = E // blk

    @functools.partial(
        pl.kernel,
        out_type=jax.ShapeDtypeStruct((2, N, HID), jnp.float32),
        mesh=_MESH,
        compiler_params=_SC_PARAMS,
        scratch_types=[
            pltpu.VMEM((blk, 16), jnp.float32),
            pltpu.VMEM((blk, 16), jnp.float32),
            pltpu.VMEM((blk, HID), jnp.float32),
            pltpu.VMEM((blk, HID), jnp.float32),
            pltpu.VMEM((blk, HID), jnp.float32),
            pltpu.VMEM((blk, HID), jnp.float32),
            pltpu.VMEM((blk, 16), jnp.float32),
            pltpu.VMEM((blk, 16), jnp.float32),
            pltpu.VMEM((blk, 16), jnp.float32),
            pltpu.VMEM((blk, 16), jnp.float32),
            pltpu.VMEM((1, blk), jnp.int32),
            pltpu.VMEM((1, blk), jnp.int32),
            pltpu.VMEM((1, blk), jnp.int32),
            pltpu.VMEM((1, blk), jnp.int32),
            pltpu.VMEM((NW, 16), jnp.float32),
            pltpu.VMEM((16,), jnp.float32),
            pltpu.VMEM_SHARED((N, HID), jnp.float32),
            pltpu.SemaphoreType.DMA,
            pltpu.SemaphoreType.DMA,
        ],
    )
    def k_(lg_h, g_h, e_h, v_h, d0_h, d1_h, dst_h, src_h, z_h, agg_h,
           lg0, lg1, ev0, ev1, vs0, vs1, d00, d01, d10, d11,
           db0, db1, sb0, sb1, gm, gv, agg_sp, sm0, sm1):
        lgs, evs, vss = (lg0, lg1), (ev0, ev1), (vs0, vs1)
        d0s, d1s = (d00, d01), (d10, d11)
        dbuf, sbuf, sems = (db0, db1), (sb0, sb1), (sm0, sm1)
        cid = lax.axis_index("c")
        sid = lax.axis_index("s")
        w = cid * 16 + sid
        pltpu.sync_copy(z_h.at[pl.ds(sid * ROWS_PER_TILE, ROWS_PER_TILE)],
                        agg_sp.at[pl.ds(sid * ROWS_PER_TILE, ROWS_PER_TILE)])
        _load_gmax(g_h, gm, gv)
        plsc.subcore_barrier()
        gvec = gv[...]
        n_iter = (nch + NW - 1) // NW

        def issue(t, p):
            pltpu.sync_copy(dst_h.at[pl.ds(t, 1)], dbuf[p])
            pltpu.sync_copy(src_h.at[pl.ds(t, 1)], sbuf[p])
            pltpu.async_copy(v_h.at[sbuf[p].at[0]], vss[p], sems[p])
            pltpu.async_copy(d0_h.at[dbuf[p].at[0]], d0s[p], sems[p])
            pltpu.async_copy(d1_h.at[dbuf[p].at[0]], d1s[p], sems[p])
            pltpu.async_copy(lg_h.at[pl.ds(t * blk, blk)], lgs[p], sems[p])
            pltpu.async_copy(e_h.at[pl.ds(t * blk, blk)], evs[p], sems[p])

        def drain(t, p):
            pltpu.make_async_copy(
                v_h.at[sbuf[p].at[0]], vss[p], sems[p]).wait()
            pltpu.make_async_copy(
                d0_h.at[dbuf[p].at[0]], d0s[p], sems[p]).wait()
            pltpu.make_async_copy(
                d1_h.at[dbuf[p].at[0]], d1s[p], sems[p]).wait()
            pltpu.make_async_copy(
                lg_h.at[pl.ds(t * blk, blk)], lgs[p], sems[p]).wait()
            pltpu.make_async_copy(
                e_h.at[pl.ds(t * blk, blk)], evs[p], sems[p]).wait()

        @pl.when(w < nch)
        def _():
            issue(w, 0)

        def body(i, carry):
            t = w + i * NW

            @pl.when(t < nch)
            def _():
                for p in range(2):

                    @pl.when(i % 2 == p)
                    def _():
                        drain(t, p)

                        @pl.when(t + NW < nch)
                        def _():
                            issue(t + NW, 1 - p)

                        lg, ev, vs, d0, d1 = (lgs[p], evs[p], vss[p],
                                              d0s[p], d1s[p])

                        def inner(b, c2):
                            exv = jnp.exp(lg[b] - gvec)
                            denv = d0[b] + d1[b] + 1e-16
                            alv = exv / denv
                            for h in range(HEADS):
                                ev[b, pl.ds(h * 16, 16)] = (
                                    vs[b, pl.ds(h * 16, 16)]
                                    + ev[b, pl.ds(h * 16, 16)]
                                ) * alv[h]
                            return c2

                        lax.fori_loop(0, blk, inner, 0)
                        pltpu.sync_copy(ev, agg_sp.at[dbuf[p].at[0]],
                                        add=True)

            return carry

        lax.fori_loop(0, n_iter, body, 0)
        plsc.subcore_barrier()
        pltpu.sync_copy(agg_sp.at[pl.ds(sid * ROWS_PER_TILE, ROWS_PER_TILE)],
                        agg_h.at[cid].at[pl.ds(sid * ROWS_PER_TILE,
                                               ROWS_PER_TILE)])

    return k_(logits, gmp, e_l, v, den0, den1, dst64, src64, zeros128)


def _tc_matmul_bias(x, w, b, blk):
    m, kdim = x.shape
    n = w.shape[1]
    grid = (m + blk - 1) // blk

    def body(x_r, w_r, b_r, o_r):
        o_r[...] = jnp.dot(x_r[...], w_r[...],
                           preferred_element_type=jnp.float32) + b_r[...]

    return pl.pallas_call(
        body,
        grid=(grid,),
        in_specs=[
            pl.BlockSpec((blk, kdim), lambda i: (i, 0)),
            pl.BlockSpec((kdim, n), lambda i: (0, 0)),
            pl.BlockSpec((1, n), lambda i: (0, 0)),
        ],
        out_specs=pl.BlockSpec((blk, n), lambda i: (i, 0)),
        out_shape=jax.ShapeDtypeStruct((m, n), jnp.float32),
    )(x, w, b)


def _tc_fold_edge_weights(W_edge, b_edge, We):
    ein = W_edge.shape[0]

    epad = ein + 8

    def body(we_r, wl_r, be_r, wc_r):
        wl = wl_r[0]
        wc = jnp.dot(we_r[...], wl, preferred_element_type=jnp.float32)
        bc = jnp.dot(be_r[...], wl, preferred_element_type=jnp.float32)
        wc_r[0] = jnp.concatenate(
            [wc, bc, jnp.zeros((epad - ein - 1, HID), jnp.float32)], axis=0)

    return pl.pallas_call(
        body,
        grid=(L,),
        in_specs=[
            pl.BlockSpec((ein, HID), lambda i: (0, 0)),
            pl.BlockSpec((1, HID, HID), lambda i: (i, 0, 0)),
            pl.BlockSpec((1, HID), lambda i: (0, 0)),
        ],
        out_specs=pl.BlockSpec((1, epad, HID), lambda i: (i, 0, 0)),
        out_shape=jax.ShapeDtypeStruct((L, epad, HID), jnp.float32),
    )(W_edge, We, b_edge[None, :])


def _tc_proj4(x, wq, wk, wv, wskip, bskip):
    blk = 1024
    grid = (N + blk - 1) // blk

    def body(x_r, wq_r, wk_r, wv_r, ws_r, bs_r, q_r, k_r, v_r, xr_r):
        xb = x_r[...]
        q_r[...] = jnp.dot(xb, wq_r[...], preferred_element_type=jnp.float32)
        k_r[...] = jnp.dot(xb, wk_r[...], preferred_element_type=jnp.float32)
        v_r[...] = jnp.dot(xb, wv_r[...], preferred_element_type=jnp.float32)
        xr_r[...] = jnp.dot(xb, ws_r[...],
                            preferred_element_type=jnp.float32) + bs_r[...]

    o = jax.ShapeDtypeStruct((N, HID), jnp.float32)
    wspec = pl.BlockSpec((HID, HID), lambda i: (0, 0))
    return pl.pallas_call(
        body,
        grid=(grid,),
        in_specs=[pl.BlockSpec((blk, HID), lambda i: (i, 0)),
                  wspec, wspec, wspec, wspec,
                  pl.BlockSpec((1, HID), lambda i: (0, 0))],
        out_specs=[pl.BlockSpec((blk, HID), lambda i: (i, 0))] * 4,
        out_shape=[o, o, o, o],
    )(x, wq, wk, wv, wskip, bskip)


def _tc_post(x, agg0, agg1, xr, wba, wbx, g1, b1, wf1, bf1, wf2, bf2, g2, b2):
    blk = 1024
    grid = (N + blk - 1) // blk

    def ln(y, g, b):
        m = jnp.mean(y, axis=-1, keepdims=True)
        v = jnp.mean((y - m) ** 2, axis=-1, keepdims=True)
        return g * (y - m) / jnp.sqrt(v + 1e-5) + b

    def body(x_r, a0_r, a1_r, xr_r, wba_r, wbx_r, g1_r, b1_r,
             wf1_r, bf1_r, wf2_r, bf2_r, g2_r, b2_r, o_r):
        agg = a0_r[...] + a1_r[...]
        xrb = xr_r[...]
        bl = (jnp.dot(agg, wba_r[...], preferred_element_type=jnp.float32)
              + jnp.dot(xrb, wbx_r[...], preferred_element_type=jnp.float32))
        beta = jax.nn.sigmoid(bl)
        h = beta * xrb + (1.0 - beta) * agg
        y = ln(x_r[...] + h, g1_r[...], b1_r[...])
        h2 = jnp.dot(
            jax.nn.gelu(jnp.dot(y, wf1_r[...],
                                preferred_element_type=jnp.float32)
                        + bf1_r[...]),
            wf2_r[...], preferred_element_type=jnp.float32) + bf2_r[...]
        o_r[...] = ln(y + h2, g2_r[...], b2_r[...])

    nblk = pl.BlockSpec((blk, HID), lambda i: (i, 0))
    row = pl.BlockSpec((1, HID), lambda i: (0, 0))
    return pl.pallas_call(
        body,
        grid=(grid,),
        in_specs=[
            nblk, nblk, nblk, nblk,
            pl.BlockSpec((HID, 1), lambda i: (0, 0)),
            pl.BlockSpec((HID, 1), lambda i: (0, 0)),
            row, row,
            pl.BlockSpec((HID, 4 * HID), lambda i: (0, 0)),
            pl.BlockSpec((1, 4 * HID), lambda i: (0, 0)),
            pl.BlockSpec((4 * HID, HID), lambda i: (0, 0)),
            row, row, row,
        ],
        out_specs=nblk,
        out_shape=jax.ShapeDtypeStruct((N, HID), jnp.float32),
    )(x, agg0, agg1, xr, wba, wbx, g1, b1, wf1, bf1, wf2, bf2, g2, b2)


def kernel(x_cont, node_cat, lookahead_cat, package_postal, edge_index,
           edge_cont, edge_cat, node_tables, lookahead_tables, edge_tables,
           postal_table, W_node, b_node, W_edge, b_edge, Wq, Wk, Wv, We,
           Wskip, bskip, Wbeta, ln1_g, ln1_b, Wf1, bf1, Wf2, bf2,
           ln2_g, ln2_b):
    i32 = jnp.int32
    f32 = jnp.float32

    stacked = jnp.concatenate([
        node_tables.reshape(-1, EMBED),
        lookahead_tables.reshape(-1, EMBED),
        postal_table,
        edge_tables.reshape(-1, EMBED),
    ], axis=0)
    offn = (jnp.arange(9, dtype=i32) * VOCAB)[None, :]
    offl = ((9 + jnp.arange(7, dtype=i32)) * VOCAB)[None, :]
    offe = ((17 + jnp.arange(9, dtype=i32)) * VOCAB)[None, :]
    idx_n = jnp.concatenate([
        node_cat.astype(i32) + offn,
        lookahead_cat.astype(i32) + offl,
        package_postal.astype(i32) + 16 * VOCAB,
    ], axis=1).reshape(-1)
    idx_e = (edge_cat.astype(i32) + offe).reshape(-1)
    idx_all = jnp.concatenate([idx_n, idx_e])
    total = idx_all.shape[0]
    n_chunks = (total + 1023) // 1024
    pad = n_chunks * 1024 - total
    idx_all = jnp.concatenate([idx_all, jnp.zeros((pad,), i32)])
    idx2d = idx_all.reshape(n_chunks * 8, 128)

    gathered = _sc_embed_gather(stacked, idx2d, n_chunks)
    node_emb = gathered[:180000].reshape(N, 18 * EMBED)
    edge_emb = gathered[180000:180000 + 9 * E].reshape(E, 9 * EMBED)

    x_in = jnp.concatenate([x_cont, node_emb], axis=1)
    x = _tc_matmul_bias(x_in, W_node, b_node[None, :], 1024)
    ef_in = jnp.concatenate([edge_cont, edge_emb], axis=1)
    folded = _tc_fold_edge_weights(W_edge, b_edge, We)

    src2d = edge_index[0].astype(i32).reshape(E // 128, 128)
    dst2d = edge_index[1].astype(i32).reshape(E // 128, 128)
    src64 = edge_index[0].astype(i32).reshape(E // 64, 64)
    dst64 = edge_index[1].astype(i32).reshape(E // 64, 64)

    z16 = jnp.zeros((N, 16), f32)
    z128 = jnp.zeros((N, HID), f32)

    for l in range(L):
        q, k, v, xr = _tc_proj4(x, Wq[l], Wk[l], Wv[l], Wskip[l],
                                bskip[l][None, :])
        e_l = _tc_matmul_bias(ef_in, folded[l, :152], folded[l, 152:153],
                              2048)
        logits, gmp = _sc_e1(q, k, e_l, dst2d, src2d)
        den = _sc_den_scatter(logits, gmp, dst2d, z16)
        agg = _sc_agg_scatter(logits, gmp, e_l, v, den[0], den[1],
                              dst64, src64, z128)
        wb = Wbeta[l]
        wba = wb[:HID] + wb[2 * HID:]
        wbx = wb[HID:2 * HID] - wb[2 * HID:]
        x = _tc_post(x, agg[0], agg[1], xr, wba, wbx,
                     ln1_g[l][None, :], ln1_b[l][None, :],
                     Wf1[l], bf1[l][None, :], Wf2[l], bf2[l][None, :],
                     ln2_g[l][None, :], ln2_b[l][None, :])
    return x


# speedup vs baseline: 1.8425x; 1.1259x over previous
import functools

import jax
import jax.numpy as jnp
import numpy as np
from jax import lax
from jax.experimental import pallas as pl
from jax.experimental.pallas import tpu as pltpu
from jax.experimental.pallas import tpu_sc as plsc

N = 10000
E = 320000
EMBED = 16
HID = 128
HEADS = 8
DH = 16
VOCAB = 20000
L = 2
NEG = -1e30
BIG = 1e30

_MESH = plsc.VectorSubcoreMesh(core_axis_name="c", subcore_axis_name="s")
_SC_PARAMS = pltpu.CompilerParams(use_tc_tiling_on_sc=False)
_SC_PARAMS_NL = pltpu.CompilerParams(use_tc_tiling_on_sc=False,
                                     needs_layout_passes=False)
NW = 32
ROWS_PER_TILE = N // 16


def _wid():
    return lax.axis_index("c") * 16 + lax.axis_index("s")


def _sc_embed_gather(table, idx2d, n_chunks):
    @functools.partial(
        pl.kernel,
        out_type=jax.ShapeDtypeStruct((n_chunks * 1024, 16), jnp.float32),
        mesh=_MESH,
        compiler_params=_SC_PARAMS,
        scratch_types=[
            pltpu.VMEM((8, 128), jnp.int32),
            pltpu.VMEM((1024, 16), jnp.float32),
            pltpu.SemaphoreType.DMA,
        ],
    )
    def k(table_h, idx_h, out_h, idx_v, rows_v, sem):
        w = _wid()
        n_iter = (n_chunks + NW - 1) // NW

        def body(i, carry):
            t = w + i * NW

            @pl.when(t < n_chunks)
            def _():
                pltpu.sync_copy(idx_h.at[pl.ds(t * 8, 8)], idx_v)
                descs = [
                    pltpu.async_copy(
                        table_h.at[idx_v.at[j]],
                        rows_v.at[pl.ds(j * 128, 128)],
                        sem,
                    )
                    for j in range(8)
                ]
                for d in descs:
                    d.wait()
                pltpu.sync_copy(rows_v, out_h.at[pl.ds(t * 1024, 1024)])

            return carry

        lax.fori_loop(0, n_iter, body, 0)

    return k(table, idx2d)


def _sc_e1(q, k, e_l, dst2d, src2d):
    nch = E // 128

    @functools.partial(
        pl.kernel,
        out_type=(
            jax.ShapeDtypeStruct((E, 16), jnp.float32),
            jax.ShapeDtypeStruct((NW, 16), jnp.float32),
        ),
        mesh=_MESH,
        compiler_params=_SC_PARAMS_NL,
        scratch_types=[
            pltpu.VMEM((1, 128), jnp.int32),
            pltpu.VMEM((1, 128), jnp.int32),
            pltpu.VMEM((1, 128), jnp.int32),
            pltpu.VMEM((1, 128), jnp.int32),
            pltpu.VMEM((128, HID), jnp.float32),
            pltpu.VMEM((128, HID), jnp.float32),
            pltpu.VMEM((128, HID), jnp.float32),
            pltpu.VMEM((128, HID), jnp.float32),
            pltpu.VMEM((128, HID), jnp.float32),
            pltpu.VMEM((128, HID), jnp.float32),
            pltpu.VMEM((128, 16), jnp.float32),
            pltpu.VMEM((8, 16), jnp.float32),
            pltpu.VMEM((16,), jnp.float32),
            pltpu.SemaphoreType.DMA,
            pltpu.SemaphoreType.DMA,
        ],
    )
    def k_(q_h, k_h, e_h, dst_h, src_h, lg_h, gmp_h,
           db0, db1, sb0, sb1, qb0, qb1, kb0, kb1, eb0, eb1,
           lg, mxh, mxb, sm0, sm1):
        dbuf, sbuf = (db0, db1), (sb0, sb1)
        qb, kb, eb = (qb0, qb1), (kb0, kb1), (eb0, eb1)
        sems = (sm0, sm1)
        w = _wid()
        n_iter = (nch + NW - 1) // NW
        iota = lax.iota(jnp.int32, 16)
        for h in range(HEADS):
            mxh[h] = jnp.full((16,), NEG, jnp.float32)

        def issue(t, p):
            pltpu.sync_copy(dst_h.at[pl.ds(t, 1)], dbuf[p])
            pltpu.sync_copy(src_h.at[pl.ds(t, 1)], sbuf[p])
            pltpu.async_copy(q_h.at[dbuf[p].at[0]], qb[p], sems[p])
            pltpu.async_copy(k_h.at[sbuf[p].at[0]], kb[p], sems[p])
            pltpu.async_copy(e_h.at[pl.ds(t * 128, 128)], eb[p], sems[p])

        def drain(t, p):
            pltpu.make_async_copy(
                q_h.at[dbuf[p].at[0]], qb[p], sems[p]).wait()
            pltpu.make_async_copy(
                k_h.at[sbuf[p].at[0]], kb[p], sems[p]).wait()
            pltpu.make_async_copy(
                e_h.at[pl.ds(t * 128, 128)], eb[p], sems[p]).wait()

        @pl.when(w < nch)
        def _():
            issue(w, 0)

        def compute(qbp, kbp, ebp):

                def grp(g, c2):
                    ri = g * 16 + iota
                    for h in range(HEADS):
                        acc = jnp.zeros((16,), jnp.float32)
                        for j in range(DH):
                            cs = h * DH + ((iota + j) & 15)
                            qv = plsc.load_gather(qbp, [ri, cs])
                            kv = plsc.load_gather(kbp, [ri, cs])
                            ev = plsc.load_gather(ebp, [ri, cs])
                            acc = acc + qv * (kv + ev)
                        acc = acc * 0.25
                        plsc.store_scatter(
                            lg, [ri, jnp.full((16,), h, jnp.int32)], acc)
                        mxh[h] = jnp.maximum(mxh[h], acc)
                    return c2

                lax.fori_loop(0, 8, grp, 0)

        def body(i, carry):
            t = w + i * NW

            @pl.when(t < nch)
            def _():
                for p in range(2):

                    @pl.when(i % 2 == p)
                    def _():
                        drain(t, p)

                        @pl.when(t + NW < nch)
                        def _():
                            issue(t + NW, 1 - p)

                        compute(qb[p], kb[p], eb[p])

                pltpu.sync_copy(lg, lg_h.at[pl.ds(t * 128, 128)])

            return carry

        lax.fori_loop(0, n_iter, body, 0)
        res = jnp.full((16,), BIG, jnp.float32)
        for h in range(HEADS):
            s = jnp.max(mxh[h])
            res = jnp.where(iota == h, s, res)
        mxb[...] = res
        pltpu.sync_copy(mxb, gmp_h.at[w])

    return k_(q, k, e_l, dst2d, src2d)


def _load_gmax(gmp_h, gm, gv):
    pltpu.sync_copy(gmp_h, gm)
    g = gm[0]
    for r in range(1, NW):
        g = jnp.maximum(g, gm[r])
    gv[...] = g


def _sc_den_scatter(logits, gmp, dst2d, zeros16):
    nch = E // 512

    @functools.partial(
        pl.kernel,
        out_type=jax.ShapeDtypeStruct((2, N, 16), jnp.float32),
        mesh=_MESH,
        compiler_params=_SC_PARAMS,
        scratch_types=[
            pltpu.VMEM((512, 16), jnp.float32),
            pltpu.VMEM((512, 16), jnp.float32),
            pltpu.VMEM((4, 128), jnp.int32),
            pltpu.VMEM((NW, 16), jnp.float32),
            pltpu.VMEM((16,), jnp.float32),
            pltpu.VMEM_SHARED((N, 16), jnp.float32),
            pltpu.SemaphoreType.DMA,
        ],
    )
    def k_(lg_h, g_h, dst_h, z_h, den_h, lg, ex, dbuf, gm, gv, den_sp, sem):
        cid = lax.axis_index("c")
        sid = lax.axis_index("s")
        w = cid * 16 + sid
        pltpu.sync_copy(z_h.at[pl.ds(sid * ROWS_PER_TILE, ROWS_PER_TILE)],
                        den_sp.at[pl.ds(sid * ROWS_PER_TILE, ROWS_PER_TILE)])
        _load_gmax(g_h, gm, gv)
        plsc.subcore_barrier()
        gvec = gv[...]
        n_iter = (nch + NW - 1) // NW

        def body(i, carry):
            t = w + i * NW

            @pl.when(t < nch)
            def _():
                pltpu.sync_copy(lg_h.at[pl.ds(t * 512, 512)], lg)
                pltpu.sync_copy(dst_h.at[pl.ds(t * 4, 4)], dbuf)

                def inner(b, c2):
                    ex[b] = jnp.exp(lg[b] - gvec)
                    return c2

                lax.fori_loop(0, 512, inner, 0)
                for j in range(4):
                    pltpu.sync_copy(ex.at[pl.ds(j * 128, 128)],
                                    den_sp.at[dbuf.at[j]], add=True)

            return carry

        lax.fori_loop(0, n_iter, body, 0)
        plsc.subcore_barrier()
        pltpu.sync_copy(den_sp.at[pl.ds(sid * ROWS_PER_TILE, ROWS_PER_TILE)],
                        den_h.at[cid].at[pl.ds(sid * ROWS_PER_TILE,
                                               ROWS_PER_TILE)])

    return k_(logits, gmp, dst2d, zeros16)


def _sc_agg_scatter(logits, gmp, e_l, v, dst64, src64, zeros128):
    blk = 64
    nch = E // blk

    @functools.partial(
        pl.kernel,
        out_type=jax.ShapeDtypeStruct((2, N, HID), jnp.float32),
        mesh=_MESH,
        compiler_params=_SC_PARAMS,
        scratch_types=[
            pltpu.VMEM((blk, 16), jnp.float32),
            pltpu.VMEM((blk, 16), jnp.float32),
            pltpu.VMEM((blk, HID), jnp.float32),
            pltpu.VMEM((blk, HID), jnp.float32),
            pltpu.VMEM((blk, HID), jnp.float32),
            pltpu.VMEM((blk, HID), jnp.float32),
            pltpu.VMEM((1, blk), jnp.int32),
            pltpu.VMEM((1, blk), jnp.int32),
            pltpu.VMEM((1, blk), jnp.int32),
            pltpu.VMEM((1, blk), jnp.int32),
            pltpu.VMEM((NW, 16), jnp.float32),
            pltpu.VMEM((16,), jnp.float32),
            pltpu.VMEM_SHARED((N, HID), jnp.float32),
            pltpu.SemaphoreType.DMA,
            pltpu.SemaphoreType.DMA,
        ],
    )
    def k_(lg_h, g_h, e_h, v_h, dst_h, src_h, z_h, agg_h,
           lg0, lg1, ev0, ev1, vs0, vs1,
           db0, db1, sb0, sb1, gm, gv, agg_sp, sm0, sm1):
        lgs, evs, vss = (lg0, lg1), (ev0, ev1), (vs0, vs1)
        dbuf, sbuf, sems = (db0, db1), (sb0, sb1), (sm0, sm1)
        cid = lax.axis_index("c")
        sid = lax.axis_index("s")
        w = cid * 16 + sid
        pltpu.sync_copy(z_h.at[pl.ds(sid * ROWS_PER_TILE, ROWS_PER_TILE)],
                        agg_sp.at[pl.ds(sid * ROWS_PER_TILE, ROWS_PER_TILE)])
        _load_gmax(g_h, gm, gv)
        plsc.subcore_barrier()
        gvec = gv[...]
        n_iter = (nch + NW - 1) // NW

        def issue(t, p):
            pltpu.sync_copy(dst_h.at[pl.ds(t, 1)], dbuf[p])
            pltpu.sync_copy(src_h.at[pl.ds(t, 1)], sbuf[p])
            pltpu.async_copy(v_h.at[sbuf[p].at[0]], vss[p], sems[p])
            pltpu.async_copy(lg_h.at[pl.ds(t * blk, blk)], lgs[p], sems[p])
            pltpu.async_copy(e_h.at[pl.ds(t * blk, blk)], evs[p], sems[p])

        def drain(t, p):
            pltpu.make_async_copy(
                v_h.at[sbuf[p].at[0]], vss[p], sems[p]).wait()
            pltpu.make_async_copy(
                lg_h.at[pl.ds(t * blk, blk)], lgs[p], sems[p]).wait()
            pltpu.make_async_copy(
                e_h.at[pl.ds(t * blk, blk)], evs[p], sems[p]).wait()

        @pl.when(w < nch)
        def _():
            issue(w, 0)

        def body(i, carry):
            t = w + i * NW

            @pl.when(t < nch)
            def _():
                for p in range(2):

                    @pl.when(i % 2 == p)
                    def _():
                        drain(t, p)

                        @pl.when(t + NW < nch)
                        def _():
                            issue(t + NW, 1 - p)

                        lg, ev, vs = lgs[p], evs[p], vss[p]

                        def inner(b, c2):
                            exv = jnp.exp(lg[b] - gvec)
                            for h in range(HEADS):
                                ev[b, pl.ds(h * 16, 16)] = (
                                    vs[b, pl.ds(h * 16, 16)]
                                    + ev[b, pl.ds(h * 16, 16)]
                                ) * exv[h]
                            return c2

                        lax.fori_loop(0, blk, inner, 0)
                        pltpu.sync_copy(ev, agg_sp.at[dbuf[p].at[0]],
                                        add=True)

            return carry

        lax.fori_loop(0, n_iter, body, 0)
        plsc.subcore_barrier()
        pltpu.sync_copy(agg_sp.at[pl.ds(sid * ROWS_PER_TILE, ROWS_PER_TILE)],
                        agg_h.at[cid].at[pl.ds(sid * ROWS_PER_TILE,
                                               ROWS_PER_TILE)])

    return k_(logits, gmp, e_l, v, dst64, src64, zeros128)


def _tc_matmul_bias(x, w, b, blk):
    m, kdim = x.shape
    n = w.shape[1]
    grid = (m + blk - 1) // blk

    def body(x_r, w_r, b_r, o_r):
        o_r[...] = jnp.dot(x_r[...], w_r[...],
                           preferred_element_type=jnp.float32) + b_r[...]

    return pl.pallas_call(
        body,
        grid=(grid,),
        in_specs=[
            pl.BlockSpec((blk, kdim), lambda i: (i, 0)),
            pl.BlockSpec((kdim, n), lambda i: (0, 0)),
            pl.BlockSpec((1, n), lambda i: (0, 0)),
        ],
        out_specs=pl.BlockSpec((blk, n), lambda i: (i, 0)),
        out_shape=jax.ShapeDtypeStruct((m, n), jnp.float32),
    )(x, w, b)


def _tc_fold_edge_weights(W_edge, b_edge, We):
    ein = W_edge.shape[0]

    epad = ein + 8

    def body(we_r, wl_r, be_r, wc_r):
        wl = wl_r[0]
        wc = jnp.dot(we_r[...], wl, preferred_element_type=jnp.float32)
        bc = jnp.dot(be_r[...], wl, preferred_element_type=jnp.float32)
        wc_r[0] = jnp.concatenate(
            [wc, bc, jnp.zeros((epad - ein - 1, HID), jnp.float32)], axis=0)

    return pl.pallas_call(
        body,
        grid=(L,),
        in_specs=[
            pl.BlockSpec((ein, HID), lambda i: (0, 0)),
            pl.BlockSpec((1, HID, HID), lambda i: (i, 0, 0)),
            pl.BlockSpec((1, HID), lambda i: (0, 0)),
        ],
        out_specs=pl.BlockSpec((1, epad, HID), lambda i: (i, 0, 0)),
        out_shape=jax.ShapeDtypeStruct((L, epad, HID), jnp.float32),
    )(W_edge, We, b_edge[None, :])


def _tc_proj4(x, wq, wk, wv, wskip, bskip):
    blk = 1024
    grid = (N + blk - 1) // blk

    def body(x_r, wq_r, wk_r, wv_r, ws_r, bs_r, q_r, k_r, v_r, xr_r):
        xb = x_r[...]
        q_r[...] = jnp.dot(xb, wq_r[...], preferred_element_type=jnp.float32)
        k_r[...] = jnp.dot(xb, wk_r[...], preferred_element_type=jnp.float32)
        v_r[...] = jnp.dot(xb, wv_r[...], preferred_element_type=jnp.float32)
        xr_r[...] = jnp.dot(xb, ws_r[...],
                            preferred_element_type=jnp.float32) + bs_r[...]

    o = jax.ShapeDtypeStruct((N, HID), jnp.float32)
    wspec = pl.BlockSpec((HID, HID), lambda i: (0, 0))
    return pl.pallas_call(
        body,
        grid=(grid,),
        in_specs=[pl.BlockSpec((blk, HID), lambda i: (i, 0)),
                  wspec, wspec, wspec, wspec,
                  pl.BlockSpec((1, HID), lambda i: (0, 0))],
        out_specs=[pl.BlockSpec((blk, HID), lambda i: (i, 0))] * 4,
        out_shape=[o, o, o, o],
    )(x, wq, wk, wv, wskip, bskip)


def _tc_post(x, agg0, agg1, den0, den1, sel16, xr, wba, wbx, g1, b1,
             wf1, bf1, wf2, bf2, g2, b2):
    blk = 1024
    grid = (N + blk - 1) // blk

    def ln(y, g, b):
        m = jnp.mean(y, axis=-1, keepdims=True)
        v = jnp.mean((y - m) ** 2, axis=-1, keepdims=True)
        return g * (y - m) / jnp.sqrt(v + 1e-5) + b

    def body(x_r, a0_r, a1_r, d0_r, d1_r, sel_r, xr_r, wba_r, wbx_r,
             g1_r, b1_r, wf1_r, bf1_r, wf2_r, bf2_r, g2_r, b2_r, o_r):
        den = jnp.dot(d0_r[...] + d1_r[...], sel_r[...],
                      preferred_element_type=jnp.float32)
        agg = (a0_r[...] + a1_r[...]) / (den + 1e-16)
        xrb = xr_r[...]
        bl = (jnp.dot(agg, wba_r[...], preferred_element_type=jnp.float32)
              + jnp.dot(xrb, wbx_r[...], preferred_element_type=jnp.float32))
        beta = jax.nn.sigmoid(bl)
        h = beta * xrb + (1.0 - beta) * agg
        y = ln(x_r[...] + h, g1_r[...], b1_r[...])
        h2 = jnp.dot(
            jax.nn.gelu(jnp.dot(y, wf1_r[...],
                                preferred_element_type=jnp.float32)
                        + bf1_r[...]),
            wf2_r[...], preferred_element_type=jnp.float32) + bf2_r[...]
        o_r[...] = ln(y + h2, g2_r[...], b2_r[...])

    nblk = pl.BlockSpec((blk, HID), lambda i: (i, 0))
    dblk = pl.BlockSpec((blk, 16), lambda i: (i, 0))
    row = pl.BlockSpec((1, HID), lambda i: (0, 0))
    return pl.pallas_call(
        body,
        grid=(grid,),
        in_specs=[
            nblk, nblk, nblk, dblk, dblk,
            pl.BlockSpec((16, HID), lambda i: (0, 0)),
            nblk,
            pl.BlockSpec((HID, 1), lambda i: (0, 0)),
            pl.BlockSpec((HID, 1), lambda i: (0, 0)),
            row, row,
            pl.BlockSpec((HID, 4 * HID), lambda i: (0, 0)),
            pl.BlockSpec((1, 4 * HID), lambda i: (0, 0)),
            pl.BlockSpec((4 * HID, HID), lambda i: (0, 0)),
            row, row, row,
        ],
        out_specs=nblk,
        out_shape=jax.ShapeDtypeStruct((N, HID), jnp.float32),
    )(x, agg0, agg1, den0, den1, sel16, xr, wba, wbx, g1, b1,
      wf1, bf1, wf2, bf2, g2, b2)


def kernel(x_cont, node_cat, lookahead_cat, package_postal, edge_index,
           edge_cont, edge_cat, node_tables, lookahead_tables, edge_tables,
           postal_table, W_node, b_node, W_edge, b_edge, Wq, Wk, Wv, We,
           Wskip, bskip, Wbeta, ln1_g, ln1_b, Wf1, bf1, Wf2, bf2,
           ln2_g, ln2_b):
    i32 = jnp.int32
    f32 = jnp.float32

    stacked = jnp.concatenate([
        node_tables.reshape(-1, EMBED),
        lookahead_tables.reshape(-1, EMBED),
        postal_table,
        edge_tables.reshape(-1, EMBED),
    ], axis=0)
    offn = (jnp.arange(9, dtype=i32) * VOCAB)[None, :]
    offl = ((9 + jnp.arange(7, dtype=i32)) * VOCAB)[None, :]
    offe = ((17 + jnp.arange(9, dtype=i32)) * VOCAB)[None, :]
    idx_n = jnp.concatenate([
        node_cat.astype(i32) + offn,
        lookahead_cat.astype(i32) + offl,
        package_postal.astype(i32) + 16 * VOCAB,
    ], axis=1).reshape(-1)
    idx_e = (edge_cat.astype(i32) + offe).reshape(-1)
    idx_all = jnp.concatenate([idx_n, idx_e])
    total = idx_all.shape[0]
    n_chunks = (total + 1023) // 1024
    pad = n_chunks * 1024 - total
    idx_all = jnp.concatenate([idx_all, jnp.zeros((pad,), i32)])
    idx2d = idx_all.reshape(n_chunks * 8, 128)

    gathered = _sc_embed_gather(stacked, idx2d, n_chunks)
    node_emb = gathered[:180000].reshape(N, 18 * EMBED)
    edge_emb = gathered[180000:180000 + 9 * E].reshape(E, 9 * EMBED)

    x_in = jnp.concatenate([x_cont, node_emb], axis=1)
    x = _tc_matmul_bias(x_in, W_node, b_node[None, :], 1024)
    ef_in = jnp.concatenate([edge_cont, edge_emb], axis=1)
    folded = _tc_fold_edge_weights(W_edge, b_edge, We)

    src2d = edge_index[0].astype(i32).reshape(E // 128, 128)
    dst2d = edge_index[1].astype(i32).reshape(E // 128, 128)
    src64 = edge_index[0].astype(i32).reshape(E // 64, 64)
    dst64 = edge_index[1].astype(i32).reshape(E // 64, 64)

    z16 = jnp.zeros((N, 16), f32)
    z128 = jnp.zeros((N, HID), f32)
    sel16 = jnp.array(np.concatenate(
        [np.repeat(np.eye(8, dtype=np.float32), DH, axis=0).T,
         np.zeros((8, HID), np.float32)], axis=0))

    for l in range(L):
        q, k, v, xr = _tc_proj4(x, Wq[l], Wk[l], Wv[l], Wskip[l],
                                bskip[l][None, :])
        e_l = _tc_matmul_bias(ef_in, folded[l, :152], folded[l, 152:153],
                              2048)
        logits, gmp = _sc_e1(q, k, e_l, dst2d, src2d)
        den = _sc_den_scatter(logits, gmp, dst2d, z16)
        agg = _sc_agg_scatter(logits, gmp, e_l, v, dst64, src64, z128)
        wb = Wbeta[l]
        wba = wb[:HID] + wb[2 * HID:]
        wbx = wb[HID:2 * HID] - wb[2 * HID:]
        x = _tc_post(x, agg[0], agg[1], den[0], den[1], sel16, xr, wba, wbx,
                     ln1_g[l][None, :], ln1_b[l][None, :],
                     Wf1[l], bf1[l][None, :], Wf2[l], bf2[l][None, :],
                     ln2_g[l][None, :], ln2_b[l][None, :])
    return x


# speedup vs baseline: 1.8648x; 1.0121x over previous
import functools

import jax
import jax.numpy as jnp
import numpy as np
from jax import lax
from jax.experimental import pallas as pl
from jax.experimental.pallas import tpu as pltpu
from jax.experimental.pallas import tpu_sc as plsc

N = 10000
E = 320000
EMBED = 16
HID = 128
HEADS = 8
DH = 16
VOCAB = 20000
L = 2
NEG = -1e30
BIG = 1e30

_MESH = plsc.VectorSubcoreMesh(core_axis_name="c", subcore_axis_name="s")
_SC_PARAMS = pltpu.CompilerParams(use_tc_tiling_on_sc=False)
_SC_PARAMS_NL = pltpu.CompilerParams(use_tc_tiling_on_sc=False,
                                     needs_layout_passes=False)
NW = 32
ROWS_PER_TILE = N // 16


def _wid():
    return lax.axis_index("c") * 16 + lax.axis_index("s")


def _sc_embed_gather(table, idx2d, n_chunks):
    @functools.partial(
        pl.kernel,
        out_type=jax.ShapeDtypeStruct((n_chunks * 1024, 16), jnp.float32),
        mesh=_MESH,
        compiler_params=_SC_PARAMS,
        scratch_types=[
            pltpu.VMEM((8, 128), jnp.int32),
            pltpu.VMEM((8, 128), jnp.int32),
            pltpu.VMEM((1024, 16), jnp.float32),
            pltpu.VMEM((1024, 16), jnp.float32),
            pltpu.SemaphoreType.DMA,
            pltpu.SemaphoreType.DMA,
        ],
    )
    def k(table_h, idx_h, out_h, iv0, iv1, rv0, rv1, sm0, sm1):
        idx_v, rows_v, sems = (iv0, iv1), (rv0, rv1), (sm0, sm1)
        w = _wid()
        n_iter = (n_chunks + NW - 1) // NW

        def issue(t, p):
            pltpu.sync_copy(idx_h.at[pl.ds(t * 8, 8)], idx_v[p])
            for j in range(8):
                pltpu.async_copy(table_h.at[idx_v[p].at[j]],
                                 rows_v[p].at[pl.ds(j * 128, 128)], sems[p])

        def drain(p):
            for j in range(8):
                pltpu.make_async_copy(
                    table_h.at[idx_v[p].at[j]],
                    rows_v[p].at[pl.ds(j * 128, 128)], sems[p]).wait()

        @pl.when(w < n_chunks)
        def _():
            issue(w, 0)

        def body(i, carry):
            t = w + i * NW

            @pl.when(t < n_chunks)
            def _():
                for p in range(2):

                    @pl.when(i % 2 == p)
                    def _():
                        drain(p)

                        @pl.when(t + NW < n_chunks)
                        def _():
                            issue(t + NW, 1 - p)

                        pltpu.sync_copy(rows_v[p],
                                        out_h.at[pl.ds(t * 1024, 1024)])

            return carry

        lax.fori_loop(0, n_iter, body, 0)

    return k(table, idx2d)


def _sc_e1(q, k, e_l, dst2d, src2d):
    nch = E // 128

    @functools.partial(
        pl.kernel,
        out_type=(
            jax.ShapeDtypeStruct((E, 16), jnp.float32),
            jax.ShapeDtypeStruct((NW, 16), jnp.float32),
        ),
        mesh=_MESH,
        compiler_params=_SC_PARAMS_NL,
        scratch_types=[
            pltpu.VMEM((1, 128), jnp.int32),
            pltpu.VMEM((1, 128), jnp.int32),
            pltpu.VMEM((1, 128), jnp.int32),
            pltpu.VMEM((1, 128), jnp.int32),
            pltpu.VMEM((128, HID), jnp.float32),
            pltpu.VMEM((128, HID), jnp.float32),
            pltpu.VMEM((128, HID), jnp.float32),
            pltpu.VMEM((128, HID), jnp.float32),
            pltpu.VMEM((128, HID), jnp.float32),
            pltpu.VMEM((128, HID), jnp.float32),
            pltpu.VMEM((128, 16), jnp.float32),
            pltpu.VMEM((8, 16), jnp.float32),
            pltpu.VMEM((16,), jnp.float32),
            pltpu.SemaphoreType.DMA,
            pltpu.SemaphoreType.DMA,
        ],
    )
    def k_(q_h, k_h, e_h, dst_h, src_h, lg_h, gmp_h,
           db0, db1, sb0, sb1, qb0, qb1, kb0, kb1, eb0, eb1,
           lg, mxh, mxb, sm0, sm1):
        dbuf, sbuf = (db0, db1), (sb0, sb1)
        qb, kb, eb = (qb0, qb1), (kb0, kb1), (eb0, eb1)
        sems = (sm0, sm1)
        w = _wid()
        n_iter = (nch + NW - 1) // NW
        iota = lax.iota(jnp.int32, 16)
        for h in range(HEADS):
            mxh[h] = jnp.full((16,), NEG, jnp.float32)

        def issue(t, p):
            pltpu.sync_copy(dst_h.at[pl.ds(t, 1)], dbuf[p])
            pltpu.sync_copy(src_h.at[pl.ds(t, 1)], sbuf[p])
            pltpu.async_copy(q_h.at[dbuf[p].at[0]], qb[p], sems[p])
            pltpu.async_copy(k_h.at[sbuf[p].at[0]], kb[p], sems[p])
            pltpu.async_copy(e_h.at[pl.ds(t * 128, 128)], eb[p], sems[p])

        def drain(t, p):
            pltpu.make_async_copy(
                q_h.at[dbuf[p].at[0]], qb[p], sems[p]).wait()
            pltpu.make_async_copy(
                k_h.at[sbuf[p].at[0]], kb[p], sems[p]).wait()
            pltpu.make_async_copy(
                e_h.at[pl.ds(t * 128, 128)], eb[p], sems[p]).wait()

        @pl.when(w < nch)
        def _():
            issue(w, 0)

        def compute(qbp, kbp, ebp):

                def grp(g, c2):
                    ri = g * 16 + iota
                    for h in range(HEADS):
                        acc = jnp.zeros((16,), jnp.float32)
                        for j in range(DH):
                            cs = h * DH + ((iota + j) & 15)
                            qv = plsc.load_gather(qbp, [ri, cs])
                            kv = plsc.load_gather(kbp, [ri, cs])
                            ev = plsc.load_gather(ebp, [ri, cs])
                            acc = acc + qv * (kv + ev)
                        acc = acc * 0.25
                        plsc.store_scatter(
                            lg, [ri, jnp.full((16,), h, jnp.int32)], acc)
                        mxh[h] = jnp.maximum(mxh[h], acc)
                    return c2

                lax.fori_loop(0, 8, grp, 0)

        def body(i, carry):
            t = w + i * NW

            @pl.when(t < nch)
            def _():
                for p in range(2):

                    @pl.when(i % 2 == p)
                    def _():
                        drain(t, p)

                        @pl.when(t + NW < nch)
                        def _():
                            issue(t + NW, 1 - p)

                        compute(qb[p], kb[p], eb[p])

                pltpu.sync_copy(lg, lg_h.at[pl.ds(t * 128, 128)])

            return carry

        lax.fori_loop(0, n_iter, body, 0)
        res = jnp.full((16,), BIG, jnp.float32)
        for h in range(HEADS):
            s = jnp.max(mxh[h])
            res = jnp.where(iota == h, s, res)
        mxb[...] = res
        pltpu.sync_copy(mxb, gmp_h.at[w])

    return k_(q, k, e_l, dst2d, src2d)


def _load_gmax(gmp_h, gm, gv):
    pltpu.sync_copy(gmp_h, gm)
    g = gm[0]
    for r in range(1, NW):
        g = jnp.maximum(g, gm[r])
    gv[...] = g


def _sc_den_scatter(logits, gmp, dst2d, zeros16):
    nch = E // 512

    @functools.partial(
        pl.kernel,
        out_type=jax.ShapeDtypeStruct((2, N, 16), jnp.float32),
        mesh=_MESH,
        compiler_params=_SC_PARAMS,
        scratch_types=[
            pltpu.VMEM((512, 16), jnp.float32),
            pltpu.VMEM((512, 16), jnp.float32),
            pltpu.VMEM((512, 16), jnp.float32),
            pltpu.VMEM((4, 128), jnp.int32),
            pltpu.VMEM((4, 128), jnp.int32),
            pltpu.VMEM((NW, 16), jnp.float32),
            pltpu.VMEM((16,), jnp.float32),
            pltpu.VMEM_SHARED((N, 16), jnp.float32),
            pltpu.SemaphoreType.DMA,
            pltpu.SemaphoreType.DMA,
        ],
    )
    def k_(lg_h, g_h, dst_h, z_h, den_h, lg0, lg1, ex, db0, db1,
           gm, gv, den_sp, sm0, sm1):
        lgs, dbuf, sems = (lg0, lg1), (db0, db1), (sm0, sm1)
        cid = lax.axis_index("c")
        sid = lax.axis_index("s")
        w = cid * 16 + sid
        pltpu.sync_copy(z_h.at[pl.ds(sid * ROWS_PER_TILE, ROWS_PER_TILE)],
                        den_sp.at[pl.ds(sid * ROWS_PER_TILE, ROWS_PER_TILE)])
        _load_gmax(g_h, gm, gv)
        plsc.subcore_barrier()
        gvec = gv[...]
        n_iter = (nch + NW - 1) // NW

        def issue(t, p):
            pltpu.sync_copy(dst_h.at[pl.ds(t * 4, 4)], dbuf[p])
            pltpu.async_copy(lg_h.at[pl.ds(t * 512, 512)], lgs[p], sems[p])

        def drain(t, p):
            pltpu.make_async_copy(
                lg_h.at[pl.ds(t * 512, 512)], lgs[p], sems[p]).wait()

        @pl.when(w < nch)
        def _():
            issue(w, 0)

        def body(i, carry):
            t = w + i * NW

            @pl.when(t < nch)
            def _():
                for p in range(2):

                    @pl.when(i % 2 == p)
                    def _():
                        drain(t, p)

                        @pl.when(t + NW < nch)
                        def _():
                            issue(t + NW, 1 - p)

                        lg = lgs[p]

                        def inner(b, c2):
                            ex[b] = jnp.exp(lg[b] - gvec)
                            return c2

                        lax.fori_loop(0, 512, inner, 0)
                        for j in range(4):
                            pltpu.sync_copy(ex.at[pl.ds(j * 128, 128)],
                                            den_sp.at[dbuf[p].at[j]],
                                            add=True)

            return carry

        lax.fori_loop(0, n_iter, body, 0)
        plsc.subcore_barrier()
        pltpu.sync_copy(den_sp.at[pl.ds(sid * ROWS_PER_TILE, ROWS_PER_TILE)],
                        den_h.at[cid].at[pl.ds(sid * ROWS_PER_TILE,
                                               ROWS_PER_TILE)])

    return k_(logits, gmp, dst2d, zeros16)


def _sc_agg_scatter(logits, gmp, e_l, v, dst64, src64, zeros128):
    blk = 64
    nch = E // blk

    @functools.partial(
        pl.kernel,
        out_type=jax.ShapeDtypeStruct((2, N, HID), jnp.float32),
        mesh=_MESH,
        compiler_params=_SC_PARAMS,
        scratch_types=[
            pltpu.VMEM((blk, 16), jnp.float32),
            pltpu.VMEM((blk, 16), jnp.float32),
            pltpu.VMEM((blk, HID), jnp.float32),
            pltpu.VMEM((blk, HID), jnp.float32),
            pltpu.VMEM((blk, HID), jnp.float32),
            pltpu.VMEM((blk, HID), jnp.float32),
            pltpu.VMEM((1, blk), jnp.int32),
            pltpu.VMEM((1, blk), jnp.int32),
            pltpu.VMEM((1, blk), jnp.int32),
            pltpu.VMEM((1, blk), jnp.int32),
            pltpu.VMEM((NW, 16), jnp.float32),
            pltpu.VMEM((16,), jnp.float32),
            pltpu.VMEM_SHARED((N, HID), jnp.float32),
            pltpu.SemaphoreType.DMA,
            pltpu.SemaphoreType.DMA,
        ],
    )
    def k_(lg_h, g_h, e_h, v_h, dst_h, src_h, z_h, agg_h,
           lg0, lg1, ev0, ev1, vs0, vs1,
           db0, db1, sb0, sb1, gm, gv, agg_sp, sm0, sm1):
        lgs, evs, vss = (lg0, lg1), (ev0, ev1), (vs0, vs1)
        dbuf, sbuf, sems = (db0, db1), (sb0, sb1), (sm0, sm1)
        cid = lax.axis_index("c")
        sid = lax.axis_index("s")
        w = cid * 16 + sid
        pltpu.sync_copy(z_h.at[pl.ds(sid * ROWS_PER_TILE, ROWS_PER_TILE)],
                        agg_sp.at[pl.ds(sid * ROWS_PER_TILE, ROWS_PER_TILE)])
        _load_gmax(g_h, gm, gv)
        plsc.subcore_barrier()
        gvec = gv[...]
        n_iter = (nch + NW - 1) // NW

        def issue(t, p):
            pltpu.sync_copy(dst_h.at[pl.ds(t, 1)], dbuf[p])
            pltpu.sync_copy(src_h.at[pl.ds(t, 1)], sbuf[p])
            pltpu.async_copy(v_h.at[sbuf[p].at[0]], vss[p], sems[p])
            pltpu.async_copy(lg_h.at[pl.ds(t * blk, blk)], lgs[p], sems[p])
            pltpu.async_copy(e_h.at[pl.ds(t * blk, blk)], evs[p], sems[p])

        def drain(t, p):
            pltpu.make_async_copy(
                v_h.at[sbuf[p].at[0]], vss[p], sems[p]).wait()
            pltpu.make_async_copy(
                lg_h.at[pl.ds(t * blk, blk)], lgs[p], sems[p]).wait()
            pltpu.make_async_copy(
                e_h.at[pl.ds(t * blk, blk)], evs[p], sems[p]).wait()

        @pl.when(w < nch)
        def _():
            issue(w, 0)

        def body(i, carry):
            t = w + i * NW

            @pl.when(t < nch)
            def _():
                for p in range(2):

                    @pl.when(i % 2 == p)
                    def _():
                        drain(t, p)

                        @pl.when(t + NW < nch)
                        def _():
                            issue(t + NW, 1 - p)

                        lg, ev, vs = lgs[p], evs[p], vss[p]

                        def inner(b, c2):
                            exv = jnp.exp(lg[b] - gvec)
                            for h in range(HEADS):
                                ev[b, pl.ds(h * 16, 16)] = (
                                    vs[b, pl.ds(h * 16, 16)]
                                    + ev[b, pl.ds(h * 16, 16)]
                                ) * exv[h]
                            return c2

                        lax.fori_loop(0, blk, inner, 0)
                        pltpu.sync_copy(ev, agg_sp.at[dbuf[p].at[0]],
                                        add=True)

            return carry

        lax.fori_loop(0, n_iter, body, 0)
        plsc.subcore_barrier()
        pltpu.sync_copy(agg_sp.at[pl.ds(sid * ROWS_PER_TILE, ROWS_PER_TILE)],
                        agg_h.at[cid].at[pl.ds(sid * ROWS_PER_TILE,
                                               ROWS_PER_TILE)])

    return k_(logits, gmp, e_l, v, dst64, src64, zeros128)


def _tc_matmul_bias(x, w, b, blk):
    m, kdim = x.shape
    n = w.shape[1]
    grid = (m + blk - 1) // blk

    def body(x_r, w_r, b_r, o_r):
        o_r[...] = jnp.dot(x_r[...], w_r[...],
                           preferred_element_type=jnp.float32) + b_r[...]

    return pl.pallas_call(
        body,
        grid=(grid,),
        in_specs=[
            pl.BlockSpec((blk, kdim), lambda i: (i, 0)),
            pl.BlockSpec((kdim, n), lambda i: (0, 0)),
            pl.BlockSpec((1, n), lambda i: (0, 0)),
        ],
        out_specs=pl.BlockSpec((blk, n), lambda i: (i, 0)),
        out_shape=jax.ShapeDtypeStruct((m, n), jnp.float32),
    )(x, w, b)


def _tc_fold_edge_weights(W_edge, b_edge, We):
    ein = W_edge.shape[0]

    epad = ein + 8

    def body(we_r, wl_r, be_r, wc_r):
        wl = wl_r[0]
        wc = jnp.dot(we_r[...], wl, preferred_element_type=jnp.float32)
        bc = jnp.dot(be_r[...], wl, preferred_element_type=jnp.float32)
        wc_r[0] = jnp.concatenate(
            [wc, bc, jnp.zeros((epad - ein - 1, HID), jnp.float32)], axis=0)

    return pl.pallas_call(
        body,
        grid=(L,),
        in_specs=[
            pl.BlockSpec((ein, HID), lambda i: (0, 0)),
            pl.BlockSpec((1, HID, HID), lambda i: (i, 0, 0)),
            pl.BlockSpec((1, HID), lambda i: (0, 0)),
        ],
        out_specs=pl.BlockSpec((1, epad, HID), lambda i: (i, 0, 0)),
        out_shape=jax.ShapeDtypeStruct((L, epad, HID), jnp.float32),
    )(W_edge, We, b_edge[None, :])


def _tc_proj4(x, wq, wk, wv, wskip, bskip):
    blk = 1024
    grid = (N + blk - 1) // blk

    def body(x_r, wq_r, wk_r, wv_r, ws_r, bs_r, q_r, k_r, v_r, xr_r):
        xb = x_r[...]
        q_r[...] = jnp.dot(xb, wq_r[...], preferred_element_type=jnp.float32)
        k_r[...] = jnp.dot(xb, wk_r[...], preferred_element_type=jnp.float32)
        v_r[...] = jnp.dot(xb, wv_r[...], preferred_element_type=jnp.float32)
        xr_r[...] = jnp.dot(xb, ws_r[...],
                            preferred_element_type=jnp.float32) + bs_r[...]

    o = jax.ShapeDtypeStruct((N, HID), jnp.float32)
    wspec = pl.BlockSpec((HID, HID), lambda i: (0, 0))
    return pl.pallas_call(
        body,
        grid=(grid,),
        in_specs=[pl.BlockSpec((blk, HID), lambda i: (i, 0)),
                  wspec, wspec, wspec, wspec,
                  pl.BlockSpec((1, HID), lambda i: (0, 0))],
        out_specs=[pl.BlockSpec((blk, HID), lambda i: (i, 0))] * 4,
        out_shape=[o, o, o, o],
    )(x, wq, wk, wv, wskip, bskip)


def _tc_post(x, agg0, agg1, den0, den1, sel16, xr, wba, wbx, g1, b1,
             wf1, bf1, wf2, bf2, g2, b2):
    blk = 1024
    grid = (N + blk - 1) // blk

    def ln(y, g, b):
        m = jnp.mean(y, axis=-1, keepdims=True)
        v = jnp.mean((y - m) ** 2, axis=-1, keepdims=True)
        return g * (y - m) / jnp.sqrt(v + 1e-5) + b

    def body(x_r, a0_r, a1_r, d0_r, d1_r, sel_r, xr_r, wba_r, wbx_r,
             g1_r, b1_r, wf1_r, bf1_r, wf2_r, bf2_r, g2_r, b2_r, o_r):
        den = jnp.dot(d0_r[...] + d1_r[...], sel_r[...],
                      preferred_element_type=jnp.float32)
        agg = (a0_r[...] + a1_r[...]) / (den + 1e-16)
        xrb = xr_r[...]
        bl = (jnp.dot(agg, wba_r[...], preferred_element_type=jnp.float32)
              + jnp.dot(xrb, wbx_r[...], preferred_element_type=jnp.float32))
        beta = jax.nn.sigmoid(bl)
        h = beta * xrb + (1.0 - beta) * agg
        y = ln(x_r[...] + h, g1_r[...], b1_r[...])
        h2 = jnp.dot(
            jax.nn.gelu(jnp.dot(y, wf1_r[...],
                                preferred_element_type=jnp.float32)
                        + bf1_r[...]),
            wf2_r[...], preferred_element_type=jnp.float32) + bf2_r[...]
        o_r[...] = ln(y + h2, g2_r[...], b2_r[...])

    nblk = pl.BlockSpec((blk, HID), lambda i: (i, 0))
    dblk = pl.BlockSpec((blk, 16), lambda i: (i, 0))
    row = pl.BlockSpec((1, HID), lambda i: (0, 0))
    return pl.pallas_call(
        body,
        grid=(grid,),
        in_specs=[
            nblk, nblk, nblk, dblk, dblk,
            pl.BlockSpec((16, HID), lambda i: (0, 0)),
            nblk,
            pl.BlockSpec((HID, 1), lambda i: (0, 0)),
            pl.BlockSpec((HID, 1), lambda i: (0, 0)),
            row, row,
            pl.BlockSpec((HID, 4 * HID), lambda i: (0, 0)),
            pl.BlockSpec((1, 4 * HID), lambda i: (0, 0)),
            pl.BlockSpec((4 * HID, HID), lambda i: (0, 0)),
            row, row, row,
        ],
        out_specs=nblk,
        out_shape=jax.ShapeDtypeStruct((N, HID), jnp.float32),
    )(x, agg0, agg1, den0, den1, sel16, xr, wba, wbx, g1, b1,
      wf1, bf1, wf2, bf2, g2, b2)


def kernel(x_cont, node_cat, lookahead_cat, package_postal, edge_index,
           edge_cont, edge_cat, node_tables, lookahead_tables, edge_tables,
           postal_table, W_node, b_node, W_edge, b_edge, Wq, Wk, Wv, We,
           Wskip, bskip, Wbeta, ln1_g, ln1_b, Wf1, bf1, Wf2, bf2,
           ln2_g, ln2_b):
    i32 = jnp.int32
    f32 = jnp.float32

    stacked = jnp.concatenate([
        node_tables.reshape(-1, EMBED),
        lookahead_tables.reshape(-1, EMBED),
        postal_table,
        edge_tables.reshape(-1, EMBED),
    ], axis=0)
    offn = (jnp.arange(9, dtype=i32) * VOCAB)[None, :]
    offl = ((9 + jnp.arange(7, dtype=i32)) * VOCAB)[None, :]
    offe = ((17 + jnp.arange(9, dtype=i32)) * VOCAB)[None, :]
    idx_n = jnp.concatenate([
        node_cat.astype(i32) + offn,
        lookahead_cat.astype(i32) + offl,
        package_postal.astype(i32) + 16 * VOCAB,
    ], axis=1).reshape(-1)
    idx_e = (edge_cat.astype(i32) + offe).reshape(-1)
    idx_all = jnp.concatenate([idx_n, idx_e])
    total = idx_all.shape[0]
    n_chunks = (total + 1023) // 1024
    pad = n_chunks * 1024 - total
    idx_all = jnp.concatenate([idx_all, jnp.zeros((pad,), i32)])
    idx2d = idx_all.reshape(n_chunks * 8, 128)

    gathered = _sc_embed_gather(stacked, idx2d, n_chunks)
    node_emb = gathered[:180000].reshape(N, 18 * EMBED)
    edge_emb = gathered[180000:180000 + 9 * E].reshape(E, 9 * EMBED)

    x_in = jnp.concatenate([x_cont, node_emb], axis=1)
    x = _tc_matmul_bias(x_in, W_node, b_node[None, :], 1024)
    ef_in = jnp.concatenate([edge_cont, edge_emb], axis=1)
    folded = _tc_fold_edge_weights(W_edge, b_edge, We)

    src2d = edge_index[0].astype(i32).reshape(E // 128, 128)
    dst2d = edge_index[1].astype(i32).reshape(E // 128, 128)
    src64 = edge_index[0].astype(i32).reshape(E // 64, 64)
    dst64 = edge_index[1].astype(i32).reshape(E // 64, 64)

    z16 = jnp.zeros((N, 16), f32)
    z128 = jnp.zeros((N, HID), f32)
    sel16 = jnp.array(np.concatenate(
        [np.repeat(np.eye(8, dtype=np.float32), DH, axis=0).T,
         np.zeros((8, HID), np.float32)], axis=0))

    for l in range(L):
        q, k, v, xr = _tc_proj4(x, Wq[l], Wk[l], Wv[l], Wskip[l],
                                bskip[l][None, :])
        e_l = _tc_matmul_bias(ef_in, folded[l, :152], folded[l, 152:153],
                              2048)
        logits, gmp = _sc_e1(q, k, e_l, dst2d, src2d)
        den = _sc_den_scatter(logits, gmp, dst2d, z16)
        agg = _sc_agg_scatter(logits, gmp, e_l, v, dst64, src64, z128)
        wb = Wbeta[l]
        wba = wb[:HID] + wb[2 * HID:]
        wbx = wb[HID:2 * HID] - wb[2 * HID:]
        x = _tc_post(x, agg[0], agg[1], den[0], den[1], sel16, xr, wba, wbx,
                     ln1_g[l][None, :], ln1_b[l][None, :],
                     Wf1[l], bf1[l][None, :], Wf2[l], bf2[l][None, :],
                     ln2_g[l][None, :], ln2_b[l][None, :])
    return x


# speedup vs baseline: 1.8659x; 1.0006x over previous
import functools

import jax
import jax.numpy as jnp
import numpy as np
from jax import lax
from jax.experimental import pallas as pl
from jax.experimental.pallas import tpu as pltpu
from jax.experimental.pallas import tpu_sc as plsc

N = 10000
E = 320000
EMBED = 16
HID = 128
HEADS = 8
DH = 16
VOCAB = 20000
L = 2
NEG = -1e30
BIG = 1e30

_MESH = plsc.VectorSubcoreMesh(core_axis_name="c", subcore_axis_name="s")
_SC_PARAMS = pltpu.CompilerParams(use_tc_tiling_on_sc=False)
_SC_PARAMS_NL = pltpu.CompilerParams(use_tc_tiling_on_sc=False,
                                     needs_layout_passes=False)
NW = 32
ROWS_PER_TILE = N // 16


def _wid():
    return lax.axis_index("c") * 16 + lax.axis_index("s")


def _sc_embed_gather(table, idx2d, n_chunks):
    @functools.partial(
        pl.kernel,
        out_type=jax.ShapeDtypeStruct((n_chunks * 1024, 16), jnp.float32),
        mesh=_MESH,
        compiler_params=_SC_PARAMS,
        scratch_types=[
            pltpu.VMEM((8, 128), jnp.int32),
            pltpu.VMEM((8, 128), jnp.int32),
            pltpu.VMEM((1024, 16), jnp.float32),
            pltpu.VMEM((1024, 16), jnp.float32),
            pltpu.SemaphoreType.DMA,
            pltpu.SemaphoreType.DMA,
        ],
    )
    def k(table_h, idx_h, out_h, iv0, iv1, rv0, rv1, sm0, sm1):
        idx_v, rows_v, sems = (iv0, iv1), (rv0, rv1), (sm0, sm1)
        w = _wid()
        n_iter = (n_chunks + NW - 1) // NW

        def issue(t, p):
            pltpu.sync_copy(idx_h.at[pl.ds(t * 8, 8)], idx_v[p])
            for j in range(8):
                pltpu.async_copy(table_h.at[idx_v[p].at[j]],
                                 rows_v[p].at[pl.ds(j * 128, 128)], sems[p])

        def drain(p):
            for j in range(8):
                pltpu.make_async_copy(
                    table_h.at[idx_v[p].at[j]],
                    rows_v[p].at[pl.ds(j * 128, 128)], sems[p]).wait()

        @pl.when(w < n_chunks)
        def _():
            issue(w, 0)

        def body(i, carry):
            t = w + i * NW

            @pl.when(t < n_chunks)
            def _():
                for p in range(2):

                    @pl.when(i % 2 == p)
                    def _():
                        drain(p)

                        @pl.when(t + NW < n_chunks)
                        def _():
                            issue(t + NW, 1 - p)

                        pltpu.sync_copy(rows_v[p],
                                        out_h.at[pl.ds(t * 1024, 1024)])

            return carry

        lax.fori_loop(0, n_iter, body, 0)

    return k(table, idx2d)


def _sc_e1(q, k, e_l, dst2d, src2d):
    nch = E // 128

    @functools.partial(
        pl.kernel,
        out_type=(
            jax.ShapeDtypeStruct((E, 16), jnp.float32),
            jax.ShapeDtypeStruct((NW, 16), jnp.float32),
        ),
        mesh=_MESH,
        compiler_params=_SC_PARAMS_NL,
        scratch_types=[
            pltpu.VMEM((1, 128), jnp.int32),
            pltpu.VMEM((1, 128), jnp.int32),
            pltpu.VMEM((1, 128), jnp.int32),
            pltpu.VMEM((1, 128), jnp.int32),
            pltpu.VMEM((128, HID), jnp.float32),
            pltpu.VMEM((128, HID), jnp.float32),
            pltpu.VMEM((128, HID), jnp.float32),
            pltpu.VMEM((128, HID), jnp.float32),
            pltpu.VMEM((128, HID), jnp.float32),
            pltpu.VMEM((128, HID), jnp.float32),
            pltpu.VMEM((128, 16), jnp.float32),
            pltpu.VMEM((8, 16), jnp.float32),
            pltpu.VMEM((16,), jnp.float32),
            pltpu.SemaphoreType.DMA,
            pltpu.SemaphoreType.DMA,
        ],
    )
    def k_(q_h, k_h, e_h, dst_h, src_h, lg_h, gmp_h,
           db0, db1, sb0, sb1, qb0, qb1, kb0, kb1, eb0, eb1,
           lg, mxh, mxb, sm0, sm1):
        dbuf, sbuf = (db0, db1), (sb0, sb1)
        qb, kb, eb = (qb0, qb1), (kb0, kb1), (eb0, eb1)
        sems = (sm0, sm1)
        w = _wid()
        n_iter = (nch + NW - 1) // NW
        iota = lax.iota(jnp.int32, 16)
        for h in range(HEADS):
            mxh[h] = jnp.full((16,), NEG, jnp.float32)

        def issue(t, p):
            pltpu.sync_copy(dst_h.at[pl.ds(t, 1)], dbuf[p])
            pltpu.sync_copy(src_h.at[pl.ds(t, 1)], sbuf[p])
            pltpu.async_copy(q_h.at[dbuf[p].at[0]], qb[p], sems[p])
            pltpu.async_copy(k_h.at[sbuf[p].at[0]], kb[p], sems[p])
            pltpu.async_copy(e_h.at[pl.ds(t * 128, 128)], eb[p], sems[p])

        def drain(t, p):
            pltpu.make_async_copy(
                q_h.at[dbuf[p].at[0]], qb[p], sems[p]).wait()
            pltpu.make_async_copy(
                k_h.at[sbuf[p].at[0]], kb[p], sems[p]).wait()
            pltpu.make_async_copy(
                e_h.at[pl.ds(t * 128, 128)], eb[p], sems[p]).wait()

        @pl.when(w < nch)
        def _():
            issue(w, 0)

        def compute(qbp, kbp, ebp):

                def grp(g, c2):
                    ri = g * 16 + iota
                    for h in range(HEADS):
                        acc = jnp.zeros((16,), jnp.float32)
                        for j in range(DH):
                            cs = h * DH + ((iota + j) & 15)
                            qv = plsc.load_gather(qbp, [ri, cs])
                            kv = plsc.load_gather(kbp, [ri, cs])
                            ev = plsc.load_gather(ebp, [ri, cs])
                            acc = acc + qv * (kv + ev)
                        acc = acc * 0.25
                        plsc.store_scatter(
                            lg, [ri, jnp.full((16,), h, jnp.int32)], acc)
                        mxh[h] = jnp.maximum(mxh[h], acc)
                    return c2

                lax.fori_loop(0, 8, grp, 0)

        def body(i, carry):
            t = w + i * NW

            @pl.when(t < nch)
            def _():
                for p in range(2):

                    @pl.when(i % 2 == p)
                    def _():
                        drain(t, p)

                        @pl.when(t + NW < nch)
                        def _():
                            issue(t + NW, 1 - p)

                        compute(qb[p], kb[p], eb[p])

                pltpu.sync_copy(lg, lg_h.at[pl.ds(t * 128, 128)])

            return carry

        lax.fori_loop(0, n_iter, body, 0)
        res = jnp.full((16,), BIG, jnp.float32)
        for h in range(HEADS):
            s = jnp.max(mxh[h])
            res = jnp.where(iota == h, s, res)
        mxb[...] = res
        pltpu.sync_copy(mxb, gmp_h.at[w])

    return k_(q, k, e_l, dst2d, src2d)


def _load_gmax(gmp_h, gm, gv):
    pltpu.sync_copy(gmp_h, gm)
    g = gm[0]
    for r in range(1, NW):
        g = jnp.maximum(g, gm[r])
    gv[...] = g


def _sc_den_scatter(logits, gmp, dst2d, zeros16):
    nch = E // 512

    @functools.partial(
        pl.kernel,
        out_type=jax.ShapeDtypeStruct((2, N, 16), jnp.float32),
        mesh=_MESH,
        compiler_params=_SC_PARAMS,
        scratch_types=[
            pltpu.VMEM((512, 16), jnp.float32),
            pltpu.VMEM((512, 16), jnp.float32),
            pltpu.VMEM((512, 16), jnp.float32),
            pltpu.VMEM((4, 128), jnp.int32),
            pltpu.VMEM((4, 128), jnp.int32),
            pltpu.VMEM((NW, 16), jnp.float32),
            pltpu.VMEM((16,), jnp.float32),
            pltpu.VMEM_SHARED((N, 16), jnp.float32),
            pltpu.SemaphoreType.DMA,
            pltpu.SemaphoreType.DMA,
        ],
    )
    def k_(lg_h, g_h, dst_h, z_h, den_h, lg0, lg1, ex, db0, db1,
           gm, gv, den_sp, sm0, sm1):
        lgs, dbuf, sems = (lg0, lg1), (db0, db1), (sm0, sm1)
        cid = lax.axis_index("c")
        sid = lax.axis_index("s")
        w = cid * 16 + sid
        pltpu.sync_copy(z_h.at[pl.ds(sid * ROWS_PER_TILE, ROWS_PER_TILE)],
                        den_sp.at[pl.ds(sid * ROWS_PER_TILE, ROWS_PER_TILE)])
        _load_gmax(g_h, gm, gv)
        plsc.subcore_barrier()
        gvec = gv[...]
        n_iter = (nch + NW - 1) // NW

        def issue(t, p):
            pltpu.sync_copy(dst_h.at[pl.ds(t * 4, 4)], dbuf[p])
            pltpu.async_copy(lg_h.at[pl.ds(t * 512, 512)], lgs[p], sems[p])

        def drain(t, p):
            pltpu.make_async_copy(
                lg_h.at[pl.ds(t * 512, 512)], lgs[p], sems[p]).wait()

        @pl.when(w < nch)
        def _():
            issue(w, 0)

        def body(i, carry):
            t = w + i * NW

            @pl.when(t < nch)
            def _():
                for p in range(2):

                    @pl.when(i % 2 == p)
                    def _():
                        drain(t, p)

                        @pl.when(t + NW < nch)
                        def _():
                            issue(t + NW, 1 - p)

                        lg = lgs[p]

                        def inner(b, c2):
                            ex[b] = jnp.exp(lg[b] - gvec)
                            return c2

                        lax.fori_loop(0, 512, inner, 0)
                        for j in range(4):
                            pltpu.sync_copy(ex.at[pl.ds(j * 128, 128)],
                                            den_sp.at[dbuf[p].at[j]],
                                            add=True)

            return carry

        lax.fori_loop(0, n_iter, body, 0)
        plsc.subcore_barrier()
        pltpu.sync_copy(den_sp.at[pl.ds(sid * ROWS_PER_TILE, ROWS_PER_TILE)],
                        den_h.at[cid].at[pl.ds(sid * ROWS_PER_TILE,
                                               ROWS_PER_TILE)])

    return k_(logits, gmp, dst2d, zeros16)


def _sc_agg_scatter(logits, gmp, e_l, v, dst64, src64, zeros128):
    blk = 64
    nch = E // blk

    @functools.partial(
        pl.kernel,
        out_type=jax.ShapeDtypeStruct((2, N, HID), jnp.float32),
        mesh=_MESH,
        compiler_params=_SC_PARAMS,
        scratch_types=[
            pltpu.VMEM((blk, 16), jnp.float32),
            pltpu.VMEM((blk, 16), jnp.float32),
            pltpu.VMEM((blk, HID), jnp.float32),
            pltpu.VMEM((blk, HID), jnp.float32),
            pltpu.VMEM((blk, HID), jnp.float32),
            pltpu.VMEM((blk, HID), jnp.float32),
            pltpu.VMEM((1, blk), jnp.int32),
            pltpu.VMEM((1, blk), jnp.int32),
            pltpu.VMEM((1, blk), jnp.int32),
            pltpu.VMEM((1, blk), jnp.int32),
            pltpu.VMEM((NW, 16), jnp.float32),
            pltpu.VMEM((16,), jnp.float32),
            pltpu.VMEM_SHARED((N, HID), jnp.float32),
            pltpu.SemaphoreType.DMA,
            pltpu.SemaphoreType.DMA,
            pltpu.SemaphoreType.DMA,
            pltpu.SemaphoreType.DMA,
        ],
    )
    def k_(lg_h, g_h, e_h, v_h, dst_h, src_h, z_h, agg_h,
           lg0, lg1, ev0, ev1, vs0, vs1,
           db0, db1, sb0, sb1, gm, gv, agg_sp, sm0, sm1, ss0, ss1):
        lgs, evs, vss = (lg0, lg1), (ev0, ev1), (vs0, vs1)
        dbuf, sbuf, sems = (db0, db1), (sb0, sb1), (sm0, sm1)
        ssems = (ss0, ss1)
        cid = lax.axis_index("c")
        sid = lax.axis_index("s")
        w = cid * 16 + sid
        pltpu.sync_copy(z_h.at[pl.ds(sid * ROWS_PER_TILE, ROWS_PER_TILE)],
                        agg_sp.at[pl.ds(sid * ROWS_PER_TILE, ROWS_PER_TILE)])
        _load_gmax(g_h, gm, gv)
        plsc.subcore_barrier()
        gvec = gv[...]
        n_iter = (nch + NW - 1) // NW

        def issue(t, p):
            pltpu.sync_copy(dst_h.at[pl.ds(t, 1)], dbuf[p])
            pltpu.sync_copy(src_h.at[pl.ds(t, 1)], sbuf[p])
            pltpu.async_copy(v_h.at[sbuf[p].at[0]], vss[p], sems[p])
            pltpu.async_copy(lg_h.at[pl.ds(t * blk, blk)], lgs[p], sems[p])
            pltpu.async_copy(e_h.at[pl.ds(t * blk, blk)], evs[p], sems[p])

        def drain(t, p):
            pltpu.make_async_copy(
                v_h.at[sbuf[p].at[0]], vss[p], sems[p]).wait()
            pltpu.make_async_copy(
                lg_h.at[pl.ds(t * blk, blk)], lgs[p], sems[p]).wait()
            pltpu.make_async_copy(
                e_h.at[pl.ds(t * blk, blk)], evs[p], sems[p]).wait()

        def wait_scatter(p):
            pltpu.make_async_copy(
                evs[p], agg_sp.at[dbuf[p].at[0]], ssems[p]).wait()

        @pl.when(w < nch)
        def _():
            issue(w, 0)

        def body(i, carry):
            t = w + i * NW

            @pl.when(t < nch)
            def _():
                for p in range(2):

                    @pl.when(i % 2 == p)
                    def _():
                        drain(t, p)
                        @pl.when(i >= 1)
                        def _():
                            wait_scatter(1 - p)

                        @pl.when(t + NW < nch)
                        def _():
                            issue(t + NW, 1 - p)

                        lg, ev, vs = lgs[p], evs[p], vss[p]

                        def inner(b, c2):
                            exv = jnp.exp(lg[b] - gvec)
                            for h in range(HEADS):
                                ev[b, pl.ds(h * 16, 16)] = (
                                    vs[b, pl.ds(h * 16, 16)]
                                    + ev[b, pl.ds(h * 16, 16)]
                                ) * exv[h]
                            return c2

                        lax.fori_loop(0, blk, inner, 0)
                        pltpu.async_copy(ev, agg_sp.at[dbuf[p].at[0]],
                                         ssems[p], add=True)

            return carry

        lax.fori_loop(0, n_iter, body, 0)
        cnt = jnp.maximum(0, (nch - w + NW - 1) // NW)

        @pl.when(cnt >= 1)
        def _():
            for p in range(2):

                @pl.when((cnt - 1) % 2 == p)
                def _():
                    wait_scatter(p)

        plsc.subcore_barrier()
        pltpu.sync_copy(agg_sp.at[pl.ds(sid * ROWS_PER_TILE, ROWS_PER_TILE)],
                        agg_h.at[cid].at[pl.ds(sid * ROWS_PER_TILE,
                                               ROWS_PER_TILE)])

    return k_(logits, gmp, e_l, v, dst64, src64, zeros128)


def _tc_matmul_bias(x, w, b, blk):
    m, kdim = x.shape
    n = w.shape[1]
    grid = (m + blk - 1) // blk

    def body(x_r, w_r, b_r, o_r):
        o_r[...] = jnp.dot(x_r[...], w_r[...],
                           preferred_element_type=jnp.float32) + b_r[...]

    return pl.pallas_call(
        body,
        grid=(grid,),
        in_specs=[
            pl.BlockSpec((blk, kdim), lambda i: (i, 0)),
            pl.BlockSpec((kdim, n), lambda i: (0, 0)),
            pl.BlockSpec((1, n), lambda i: (0, 0)),
        ],
        out_specs=pl.BlockSpec((blk, n), lambda i: (i, 0)),
        out_shape=jax.ShapeDtypeStruct((m, n), jnp.float32),
    )(x, w, b)


def _tc_fold_edge_weights(W_edge, b_edge, We):
    ein = W_edge.shape[0]

    epad = ein + 8

    def body(we_r, wl_r, be_r, wc_r):
        wl = wl_r[0]
        wc = jnp.dot(we_r[...], wl, preferred_element_type=jnp.float32)
        bc = jnp.dot(be_r[...], wl, preferred_element_type=jnp.float32)
        wc_r[0] = jnp.concatenate(
            [wc, bc, jnp.zeros((epad - ein - 1, HID), jnp.float32)], axis=0)

    return pl.pallas_call(
        body,
        grid=(L,),
        in_specs=[
            pl.BlockSpec((ein, HID), lambda i: (0, 0)),
            pl.BlockSpec((1, HID, HID), lambda i: (i, 0, 0)),
            pl.BlockSpec((1, HID), lambda i: (0, 0)),
        ],
        out_specs=pl.BlockSpec((1, epad, HID), lambda i: (i, 0, 0)),
        out_shape=jax.ShapeDtypeStruct((L, epad, HID), jnp.float32),
    )(W_edge, We, b_edge[None, :])


def _tc_proj4(x, wq, wk, wv, wskip, bskip):
    blk = 1024
    grid = (N + blk - 1) // blk

    def body(x_r, wq_r, wk_r, wv_r, ws_r, bs_r, q_r, k_r, v_r, xr_r):
        xb = x_r[...]
        q_r[...] = jnp.dot(xb, wq_r[...], preferred_element_type=jnp.float32)
        k_r[...] = jnp.dot(xb, wk_r[...], preferred_element_type=jnp.float32)
        v_r[...] = jnp.dot(xb, wv_r[...], preferred_element_type=jnp.float32)
        xr_r[...] = jnp.dot(xb, ws_r[...],
                            preferred_element_type=jnp.float32) + bs_r[...]

    o = jax.ShapeDtypeStruct((N, HID), jnp.float32)
    wspec = pl.BlockSpec((HID, HID), lambda i: (0, 0))
    return pl.pallas_call(
        body,
        grid=(grid,),
        in_specs=[pl.BlockSpec((blk, HID), lambda i: (i, 0)),
                  wspec, wspec, wspec, wspec,
                  pl.BlockSpec((1, HID), lambda i: (0, 0))],
        out_specs=[pl.BlockSpec((blk, HID), lambda i: (i, 0))] * 4,
        out_shape=[o, o, o, o],
    )(x, wq, wk, wv, wskip, bskip)


def _tc_post(x, agg0, agg1, den0, den1, sel16, xr, wba, wbx, g1, b1,
             wf1, bf1, wf2, bf2, g2, b2):
    blk = 1024
    grid = (N + blk - 1) // blk

    def ln(y, g, b):
        m = jnp.mean(y, axis=-1, keepdims=True)
        v = jnp.mean((y - m) ** 2, axis=-1, keepdims=True)
        return g * (y - m) / jnp.sqrt(v + 1e-5) + b

    def body(x_r, a0_r, a1_r, d0_r, d1_r, sel_r, xr_r, wba_r, wbx_r,
             g1_r, b1_r, wf1_r, bf1_r, wf2_r, bf2_r, g2_r, b2_r, o_r):
        den = jnp.dot(d0_r[...] + d1_r[...], sel_r[...],
                      preferred_element_type=jnp.float32)
        agg = (a0_r[...] + a1_r[...]) / (den + 1e-16)
        xrb = xr_r[...]
        bl = (jnp.dot(agg, wba_r[...], preferred_element_type=jnp.float32)
              + jnp.dot(xrb, wbx_r[...], preferred_element_type=jnp.float32))
        beta = jax.nn.sigmoid(bl)
        h = beta * xrb + (1.0 - beta) * agg
        y = ln(x_r[...] + h, g1_r[...], b1_r[...])
        h2 = jnp.dot(
            jax.nn.gelu(jnp.dot(y, wf1_r[...],
                                preferred_element_type=jnp.float32)
                        + bf1_r[...]),
            wf2_r[...], preferred_element_type=jnp.float32) + bf2_r[...]
        o_r[...] = ln(y + h2, g2_r[...], b2_r[...])

    nblk = pl.BlockSpec((blk, HID), lambda i: (i, 0))
    dblk = pl.BlockSpec((blk, 16), lambda i: (i, 0))
    row = pl.BlockSpec((1, HID), lambda i: (0, 0))
    return pl.pallas_call(
        body,
        grid=(grid,),
        in_specs=[
            nblk, nblk, nblk, dblk, dblk,
            pl.BlockSpec((16, HID), lambda i: (0, 0)),
            nblk,
            pl.BlockSpec((HID, 1), lambda i: (0, 0)),
            pl.BlockSpec((HID, 1), lambda i: (0, 0)),
            row, row,
            pl.BlockSpec((HID, 4 * HID), lambda i: (0, 0)),
            pl.BlockSpec((1, 4 * HID), lambda i: (0, 0)),
            pl.BlockSpec((4 * HID, HID), lambda i: (0, 0)),
            row, row, row,
        ],
        out_specs=nblk,
        out_shape=jax.ShapeDtypeStruct((N, HID), jnp.float32),
    )(x, agg0, agg1, den0, den1, sel16, xr, wba, wbx, g1, b1,
      wf1, bf1, wf2, bf2, g2, b2)


def kernel(x_cont, node_cat, lookahead_cat, package_postal, edge_index,
           edge_cont, edge_cat, node_tables, lookahead_tables, edge_tables,
           postal_table, W_node, b_node, W_edge, b_edge, Wq, Wk, Wv, We,
           Wskip, bskip, Wbeta, ln1_g, ln1_b, Wf1, bf1, Wf2, bf2,
           ln2_g, ln2_b):
    i32 = jnp.int32
    f32 = jnp.float32

    stacked = jnp.concatenate([
        node_tables.reshape(-1, EMBED),
        lookahead_tables.reshape(-1, EMBED),
        postal_table,
        edge_tables.reshape(-1, EMBED),
    ], axis=0)
    offn = (jnp.arange(9, dtype=i32) * VOCAB)[None, :]
    offl = ((9 + jnp.arange(7, dtype=i32)) * VOCAB)[None, :]
    offe = ((17 + jnp.arange(9, dtype=i32)) * VOCAB)[None, :]
    idx_n = jnp.concatenate([
        node_cat.astype(i32) + offn,
        lookahead_cat.astype(i32) + offl,
        package_postal.astype(i32) + 16 * VOCAB,
    ], axis=1).reshape(-1)
    idx_e = (edge_cat.astype(i32) + offe).reshape(-1)
    idx_all = jnp.concatenate([idx_n, idx_e])
    total = idx_all.shape[0]
    n_chunks = (total + 1023) // 1024
    pad = n_chunks * 1024 - total
    idx_all = jnp.concatenate([idx_all, jnp.zeros((pad,), i32)])
    idx2d = idx_all.reshape(n_chunks * 8, 128)

    gathered = _sc_embed_gather(stacked, idx2d, n_chunks)
    node_emb = gathered[:180000].reshape(N, 18 * EMBED)
    edge_emb = gathered[180000:180000 + 9 * E].reshape(E, 9 * EMBED)

    x_in = jnp.concatenate([x_cont, node_emb], axis=1)
    x = _tc_matmul_bias(x_in, W_node, b_node[None, :], 1024)
    ef_in = jnp.concatenate([edge_cont, edge_emb], axis=1)
    folded = _tc_fold_edge_weights(W_edge, b_edge, We)

    src2d = edge_index[0].astype(i32).reshape(E // 128, 128)
    dst2d = edge_index[1].astype(i32).reshape(E // 128, 128)
    src64 = edge_index[0].astype(i32).reshape(E // 64, 64)
    dst64 = edge_index[1].astype(i32).reshape(E // 64, 64)

    z16 = jnp.zeros((N, 16), f32)
    z128 = jnp.zeros((N, HID), f32)
    sel16 = jnp.array(np.concatenate(
        [np.repeat(np.eye(8, dtype=np.float32), DH, axis=0).T,
         np.zeros((8, HID), np.float32)], axis=0))

    for l in range(L):
        q, k, v, xr = _tc_proj4(x, Wq[l], Wk[l], Wv[l], Wskip[l],
                                bskip[l][None, :])
        e_l = _tc_matmul_bias(ef_in, folded[l, :152], folded[l, 152:153],
                              2048)
        logits, gmp = _sc_e1(q, k, e_l, dst2d, src2d)
        den = _sc_den_scatter(logits, gmp, dst2d, z16)
        agg = _sc_agg_scatter(logits, gmp, e_l, v, dst64, src64, z128)
        wb = Wbeta[l]
        wba = wb[:HID] + wb[2 * HID:]
        wbx = wb[HID:2 * HID] - wb[2 * HID:]
        x = _tc_post(x, agg[0], agg[1], den[0], den[1], sel16, xr, wba, wbx,
                     ln1_g[l][None, :], ln1_b[l][None, :],
                     Wf1[l], bf1[l][None, :], Wf2[l], bf2[l][None, :],
                     ln2_g[l][None, :], ln2_b[l][None, :])
    return x
